# baseline (speedup 1.0000x reference)
import functools

import jax
import jax.numpy as jnp
from jax import lax
from jax.experimental import pallas as pl
from jax.experimental.pallas import tpu as pltpu
from jax.experimental.pallas import tpu_sc as plsc

F32 = jnp.float32

NC = 2
NS = 16
NW = NC * NS


def _cond_body(src_ref, srcs_ref, dst_ref, dsts_ref,
               vx_ref, vy_ref, vz_ref, vxs_ref, vys_ref, vzs_ref,
               aw1_ref, ab1_ref, ag_ref, abt_ref,
               mask_ref, o_ref):
    s = src_ref[...]
    ss = srcs_ref[...]
    d = dst_ref[...]
    ds = dsts_ref[...]
    cond = (s == ss) & (s != d) & (ss != ds)
    rows, cols = cond.shape
    idx = (lax.broadcasted_iota(jnp.int32, (rows, cols), 0) * cols
           + lax.broadcasted_iota(jnp.int32, (rows, cols), 1))
    last = jnp.max(jnp.where(cond, idx, -1))
    m = (cond & (idx != last)).astype(F32)
    mask_ref[...] = m
    col0 = m[:, 0:1]
    col0_up = jnp.concatenate([col0[1:], jnp.zeros((1, 1), F32)], axis=0)
    mn = jnp.concatenate([m[:, 1:], col0_up], axis=1)

    vx = vx_ref[...]
    vy = vy_ref[...]
    vz = vz_ref[...]
    vxs = vxs_ref[...]
    vys = vys_ref[...]
    vzs = vzs_ref[...]
    nv = jnp.sqrt(vx * vx + vy * vy + vz * vz)
    nvs = jnp.sqrt(vxs * vxs + vys * vys + vzs * vzs)
    dot = vx * vxs + vy * vys + vz * vzs
    cos = (m * mn * dot
           / (jnp.maximum(nv, 1e-12) * jnp.maximum(nvs, 1e-12)))

    nf = 16
    acc = jnp.zeros_like(cos)
    for j in range(nf):
        t = cos * aw1_ref[0, j] + ab1_ref[0, j]
        o_ref[j] = t
        acc = acc + t
    mu = acc * (1.0 / nf)
    acc = jnp.zeros_like(cos)
    for j in range(nf):
        dj = o_ref[j] - mu
        acc = acc + dj * dj
    inv = lax.rsqrt(acc * (1.0 / nf) + 1e-6)
    for j in range(nf):
        u = (o_ref[j] - mu) * inv * ag_ref[0, j] + abt_ref[0, j]
        o_ref[j] = u * jax.nn.sigmoid(u)


def _make_cond(src2d, srcs2d, dst2d, dsts2d, v2d, vs2d, aw1, ab1, ag, abt):
    nrow, ncol = src2d.shape
    return pl.pallas_call(
        _cond_body,
        grid=(1,),
        in_specs=([pl.BlockSpec((nrow, ncol), lambda i: (0, 0))] * 10
                  + [pl.BlockSpec(memory_space=pltpu.SMEM)] * 4),
        out_specs=[
            pl.BlockSpec((nrow, ncol), lambda i: (0, 0)),
            pl.BlockSpec((16, nrow, ncol), lambda i: (0, 0, 0)),
        ],
        out_shape=[
            jax.ShapeDtypeStruct((nrow, ncol), F32),
            jax.ShapeDtypeStruct((16, nrow, ncol), F32),
        ],
    )(src2d, srcs2d, dst2d, dsts2d, *v2d, *vs2d, aw1, ab1, ag, abt)


def _tables_body(atom_ref, wp_ref, wq_ref, wr_ref, p_ref, q_ref, r_ref):
    at = atom_ref[...]
    p_ref[...] = jnp.dot(at, wp_ref[...], preferred_element_type=F32)
    q_ref[...] = jnp.dot(at, wq_ref[...], preferred_element_type=F32)
    r_ref[...] = jnp.dot(at, wr_ref[...], preferred_element_type=F32)


def _make_tables(atom, wp, wq, wr, bn):
    n, a = atom.shape
    w = wp.shape[1]
    return pl.pallas_call(
        _tables_body,
        grid=(n // bn,),
        in_specs=[
            pl.BlockSpec((bn, a), lambda i: (i, 0)),
            pl.BlockSpec((a, w), lambda i: (0, 0)),
            pl.BlockSpec((a, w), lambda i: (0, 0)),
            pl.BlockSpec((a, w), lambda i: (0, 0)),
        ],
        out_specs=[pl.BlockSpec((bn, w), lambda i: (i, 0))] * 3,
        out_shape=[jax.ShapeDtypeStruct((n, w), F32)] * 3,
    )(atom, wp, wq, wr)


def _make_gather_sum(width, e, chunk, nbuf=2):
    per_tile = e // NW
    iters = per_tile // chunk
    mesh = plsc.VectorSubcoreMesh(core_axis_name="c", subcore_axis_name="s",
                                  num_cores=NC, num_subcores=NS)

    @functools.partial(
        pl.kernel,
        out_type=jax.ShapeDtypeStruct((e, width), F32),
        mesh=mesh,
        scratch_types=(
            [pltpu.VMEM((per_tile,), jnp.int32)] * 3
            + [pltpu.VMEM((chunk, width), F32)] * (3 * nbuf)
            + [pltpu.SemaphoreType.DMA] * (2 * nbuf)
        ),
    )
    def gsum_kernel(*refs):
        tables = refs[:3]
        idxs = refs[3:6]
        out = refs[6]
        scr = refs[7:]
        idx_all = scr[:3]
        bufs = [scr[3 + 3 * j:3 + 3 * j + 3] for j in range(nbuf)]
        gsem = scr[3 + 3 * nbuf:3 + 4 * nbuf]
        wsem = scr[3 + 4 * nbuf:3 + 5 * nbuf]
        wid = lax.axis_index("s") * NC + lax.axis_index("c")
        base0 = pl.multiple_of(wid * per_tile, 8)

        for t in range(3):
            pltpu.sync_copy(idxs[t].at[pl.ds(base0, per_tile)], idx_all[t])

        def isl(t, k):
            return idx_all[t].at[pl.ds(pl.multiple_of(k * chunk, 8), chunk)]

        def start(k, j):
            for t in range(3):
                pltpu.async_copy(tables[t].at[isl(t, k)], bufs[j][t],
                                 gsem[j])

        def wait_gathers(k, j):
            for t in range(3):
                pltpu.make_async_copy(tables[t].at[isl(t, k)], bufs[j][t],
                                      gsem[j]).wait()

        def tec_sum(j):
            b0, b1, b2 = bufs[j]

            def srow(r, _):
                for c in range(width // 16):
                    cs = pl.ds(c * 16, 16)
                    b0[r, cs] = b0[r, cs] + b1[r, cs] + b2[r, cs]
                return 0

            lax.fori_loop(0, chunk, srow, 0, unroll=False)

        def start_wb(k, j):
            b = pl.multiple_of(base0 + k * chunk, 8)
            pltpu.async_copy(bufs[j][0], out.at[pl.ds(b, chunk)], wsem[j])

        def wait_wb(k, j):
            b = pl.multiple_of(base0 + k * chunk, 8)
            pltpu.make_async_copy(bufs[j][0], out.at[pl.ds(b, chunk)],
                                  wsem[j]).wait()

        for j in range(min(nbuf, iters)):
            start(j, j)

        def body(it, _):
            for j in range(nbuf):
                k = it * nbuf + j

                @pl.when(k < iters)
                def _():
                    wait_gathers(k, j)
                    tec_sum(j)
                    start_wb(k, j)

                    @pl.when(k + nbuf < iters)
                    def _():
                        wait_wb(k, j)
                        start(k + nbuf, j)
            return 0

        lax.fori_loop(0, (iters + nbuf - 1) // nbuf, body, 0, unroll=False)
        for j in range(min(nbuf, iters)):
            klast = iters - 1 - ((iters - 1 - j) % nbuf)
            wait_wb(klast, j)

    return gsum_kernel


def _make_gather_pair(hm, e, chunk, nbuf=2):
    width = 128
    per_tile = e // NW
    iters = per_tile // chunk
    mesh = plsc.VectorSubcoreMesh(core_axis_name="c", subcore_axis_name="s",
                                  num_cores=NC, num_subcores=NS)

    @functools.partial(
        pl.kernel,
        out_type=jax.ShapeDtypeStruct((e, width), F32),
        mesh=mesh,
        scratch_types=(
            [pltpu.VMEM((per_tile,), jnp.int32)] * 2
            + [pltpu.VMEM((chunk, width), F32)] * (2 * nbuf)
            + [pltpu.SemaphoreType.DMA] * (2 * nbuf)
        ),
    )
    def gpair_kernel(*refs):
        tables = refs[:2]
        idxs = refs[2:4]
        out = refs[4]
        scr = refs[5:]
        idx_all = scr[:2]
        bufs = [scr[2 + 2 * j:2 + 2 * j + 2] for j in range(nbuf)]
        gsem = scr[2 + 2 * nbuf:2 + 3 * nbuf]
        wsem = scr[2 + 3 * nbuf:2 + 4 * nbuf]
        wid = lax.axis_index("s") * NC + lax.axis_index("c")
        base0 = pl.multiple_of(wid * per_tile, 8)

        for t in range(2):
            pltpu.sync_copy(idxs[t].at[pl.ds(base0, per_tile)], idx_all[t])

        def isl(t, k):
            return idx_all[t].at[pl.ds(pl.multiple_of(k * chunk, 8), chunk)]

        def start(k, j):
            for t in range(2):
                pltpu.async_copy(tables[t].at[isl(t, k)], bufs[j][t],
                                 gsem[j])

        def wait_gathers(k, j):
            for t in range(2):
                pltpu.make_async_copy(tables[t].at[isl(t, k)], bufs[j][t],
                                      gsem[j]).wait()

        def tec_pair(j):
            b0, b1 = bufs[j]

            def srow(r, _):
                for c in range(hm // 16):
                    cs = pl.ds(hm + c * 16, 16)
                    b0[r, cs] = b1[r, cs]
                return 0

            lax.fori_loop(0, chunk, srow, 0, unroll=False)

        def start_wb(k, j):
            b = pl.multiple_of(base0 + k * chunk, 8)
            pltpu.async_copy(bufs[j][0], out.at[pl.ds(b, chunk)], wsem[j])

        def wait_wb(k, j):
            b = pl.multiple_of(base0 + k * chunk, 8)
            pltpu.make_async_copy(bufs[j][0], out.at[pl.ds(b, chunk)],
                                  wsem[j]).wait()

        for j in range(min(nbuf, iters)):
            start(j, j)

        def body(it, _):
            for j in range(nbuf):
                k = it * nbuf + j

                @pl.when(k < iters)
                def _():
                    wait_gathers(k, j)
                    tec_pair(j)
                    start_wb(k, j)

                    @pl.when(k + nbuf < iters)
                    def _():
                        wait_wb(k, j)
                        start(k + nbuf, j)
            return 0

        lax.fori_loop(0, (iters + nbuf - 1) // nbuf, body, 0, unroll=False)
        for j in range(min(nbuf, iters)):
            klast = iters - 1 - ((iters - 1 - j) % nbuf)
            wait_wb(klast, j)

    return gpair_kernel


def _make_gather(n_tables, width, e, chunk, nbuf=4, dtype=F32):
    per_tile = e // NW
    iters = per_tile // chunk
    mesh = plsc.VectorSubcoreMesh(core_axis_name="c", subcore_axis_name="s",
                                  num_cores=NC, num_subcores=NS)

    @functools.partial(
        pl.kernel,
        out_type=[jax.ShapeDtypeStruct((e, width), dtype)] * n_tables,
        mesh=mesh,
        scratch_types=(
            [pltpu.VMEM((per_tile,), jnp.int32)] * n_tables
            + [pltpu.VMEM((chunk, width), dtype)] * nbuf
            + [pltpu.SemaphoreType.DMA] * (2 * nbuf)
        ),
    )
    def gather_kernel(*refs):
        tables = refs[:n_tables]
        idxs = refs[n_tables:2 * n_tables]
        outs = refs[2 * n_tables:3 * n_tables]
        scr = refs[3 * n_tables:]
        idx_all = scr[:n_tables]
        rows_v = scr[n_tables:n_tables + nbuf]
        gsem = scr[n_tables + nbuf:n_tables + 2 * nbuf]
        wsem = scr[n_tables + 2 * nbuf:n_tables + 3 * nbuf]
        wid = lax.axis_index("s") * NC + lax.axis_index("c")
        base0 = pl.multiple_of(wid * per_tile, 8)

        for t in range(n_tables):
            pltpu.sync_copy(idxs[t].at[pl.ds(base0, per_tile)], idx_all[t])

        for t, (tbl, out) in enumerate(zip(tables, outs)):
            iall = idx_all[t]

            def islice(k):
                return iall.at[pl.ds(pl.multiple_of(k * chunk, 8), chunk)]

            def start(k, j):
                pltpu.async_copy(tbl.at[islice(k)], rows_v[j], gsem[j])

            def wait_gather(k, j):
                pltpu.make_async_copy(tbl.at[islice(k)], rows_v[j],
                                      gsem[j]).wait()

            def start_wb(k, j):
                b = pl.multiple_of(base0 + k * chunk, 8)
                pltpu.async_copy(rows_v[j], out.at[pl.ds(b, chunk)], wsem[j])

            def wait_wb(k, j):
                b = pl.multiple_of(base0 + k * chunk, 8)
                pltpu.make_async_copy(rows_v[j], out.at[pl.ds(b, chunk)],
                                      wsem[j]).wait()

            for j in range(min(nbuf, iters)):
                start(j, j)

            def body(it, _):
                for j in range(nbuf):
                    k = it * nbuf + j

                    @pl.when(k < iters)
                    def _():
                        wait_gather(k, j)
                        start_wb(k, j)

                        @pl.when(k + nbuf < iters)
                        def _():
                            wait_wb(k, j)
                            start(k + nbuf, j)
                return 0

            lax.fori_loop(0, (iters + nbuf - 1) // nbuf, body, 0,
                          unroll=False)
            for j in range(min(nbuf, iters)):
                klast = iters - 1 - ((iters - 1 - j) % nbuf)
                wait_wb(klast, j)

    return gather_kernel


def _make_scatter_add(n, width, e, chunk):
    per_core = e // NC
    per_tile = per_core // NS
    iters = per_tile // chunk
    nrows_tile = n // NS
    mesh = plsc.VectorSubcoreMesh(core_axis_name="c", subcore_axis_name="s",
                                  num_cores=NC, num_subcores=NS)

    nbuf = 2

    @functools.partial(
        pl.kernel,
        out_type=jax.ShapeDtypeStruct((NC, n, width), F32),
        mesh=mesh,
        scratch_types=(
            [pltpu.VMEM((per_tile,), jnp.int32)]
            + [pltpu.VMEM((chunk, width), F32)] * nbuf
            + [pltpu.VMEM_SHARED((n, width), F32)]
            + [pltpu.SemaphoreType.DMA] * (2 * nbuf)
        ),
    )
    def scatter_kernel(x_hbm, dst_hbm, zeros_hbm, out_hbm,
                       idx_all, rb0, rb1, acc_sh, rs0, rs1, ss0, ss1):
        rbuf = (rb0, rb1)
        rsem = (rs0, rs1)
        ssem = (ss0, ss1)
        cid = lax.axis_index("c")
        sid = lax.axis_index("s")
        r0 = pl.multiple_of(sid * nrows_tile, 8)
        pltpu.sync_copy(zeros_hbm.at[pl.ds(r0, nrows_tile)],
                        acc_sh.at[pl.ds(r0, nrows_tile)])
        base0 = pl.multiple_of(cid * per_core + sid * per_tile, 8)
        pltpu.sync_copy(dst_hbm.at[pl.ds(base0, per_tile)], idx_all)
        plsc.subcore_barrier()

        def islice(k):
            return idx_all.at[pl.ds(pl.multiple_of(k * chunk, 8), chunk)]

        if True:
            def start_read(k, j):
                b = pl.multiple_of(base0 + k * chunk, 8)
                pltpu.async_copy(x_hbm.at[pl.ds(b, chunk)], rbuf[j], rsem[j])

            def wait_read(k, j):
                b = pl.multiple_of(base0 + k * chunk, 8)
                pltpu.make_async_copy(x_hbm.at[pl.ds(b, chunk)], rbuf[j],
                                      rsem[j]).wait()

            def start_scat(k, j):
                pltpu.async_copy(rbuf[j], acc_sh.at[islice(k)], ssem[j],
                                 add=True)

            def wait_scat(k, j):
                pltpu.make_async_copy(rbuf[j], acc_sh.at[islice(k)],
                                      ssem[j]).wait()

            for j in range(min(nbuf, iters)):
                start_read(j, j)

            def body(it, _):
                for j in range(nbuf):
                    k = it * nbuf + j

                    @pl.when(k < iters)
                    def _():
                        wait_read(k, j)
                        start_scat(k, j)

                        @pl.when(k + nbuf < iters)
                        def _():
                            wait_scat(k, j)
                            start_read(k + nbuf, j)
                return 0

            lax.fori_loop(0, (iters + nbuf - 1) // nbuf, body, 0,
                          unroll=False)
            for j in range(min(nbuf, iters)):
                klast = iters - 1 - ((iters - 1 - j) % nbuf)
                wait_scat(klast, j)

        plsc.subcore_barrier()
        pltpu.sync_copy(acc_sh.at[pl.ds(r0, nrows_tile)],
                        out_hbm.at[cid].at[pl.ds(r0, nrows_tile)])

    return scatter_kernel


def _edge_body(z_ref, ele_ref, eles_ref,
               m_ref, ang_ref,
               we_ref, wes_ref, wan_ref,
               ib1_ref, ig_ref, ibt_ref,
               out_ref):
    b = m_ref.shape[0]
    hw = we_ref.shape[1]
    m = m_ref[...]

    lin = (z_ref[...][:, :hw]
           + jnp.dot(ele_ref[...], we_ref[...], preferred_element_type=F32)
           + jnp.dot(eles_ref[...], wes_ref[...], preferred_element_type=F32)
           + jnp.dot(ang_ref[...], wan_ref[...], preferred_element_type=F32))
    h = m * lin + ib1_ref[...]
    jmat = jnp.full((hw, hw), 1.0 / hw, F32)
    mu = jnp.dot(h, jmat, preferred_element_type=F32)
    s2 = jnp.dot(h * h, jmat, preferred_element_type=F32)
    var = jnp.maximum(s2 - mu * mu, 0.0)
    h = (h - mu) * lax.rsqrt(var + 1e-6) * ig_ref[...] + ibt_ref[...]
    h = h * jax.nn.sigmoid(h)
    out_ref[...] = jnp.concatenate(
        [h, jnp.ones((b, 1), F32), jnp.zeros((b, 127 - hw), F32)], axis=1)


def _node_body(pa_ref, pb_ref, iw2_ref, ib2_ref, ms_ref, md_ref,
               inv_ref, st_ref):
    hw = iw2_ref.shape[0]
    hm = ms_ref.shape[1]
    b = pa_ref.shape[0]
    acc = pa_ref[...] + pb_ref[...]
    h2s = acc[:, :hw]
    cnt = acc[:, hw:hw + 1]
    inv = (jnp.dot(h2s, iw2_ref[...], preferred_element_type=F32)
           + cnt * ib2_ref[...])
    inv_ref[...] = inv
    st_ref[...] = jnp.concatenate(
        [jnp.dot(inv, ms_ref[...], preferred_element_type=F32),
         jnp.dot(inv, md_ref[...], preferred_element_type=F32),
         jnp.zeros((b, 128 - 2 * hm), F32)], axis=1)


def _merge_body(z5_ref, ele_ref, me_ref,
                mb1_ref, mg_ref, mbt_ref, mw2_ref, mb2_ref, out_ref):
    hm = me_ref.shape[1]
    z5 = z5_ref[...]
    h = (z5[:, :hm] + z5[:, hm:2 * hm]
         + jnp.dot(ele_ref[...], me_ref[...], preferred_element_type=F32)
         + mb1_ref[...])
    jmat = jnp.full((hm, hm), 1.0 / hm, F32)
    mu = jnp.dot(h, jmat, preferred_element_type=F32)
    s2 = jnp.dot(h * h, jmat, preferred_element_type=F32)
    var = jnp.maximum(s2 - mu * mu, 0.0)
    h = (h - mu) * lax.rsqrt(var + 1e-6) * mg_ref[...] + mbt_ref[...]
    h = h * jax.nn.sigmoid(h)
    out_ref[...] = (jnp.dot(h, mw2_ref[...], preferred_element_type=F32)
                    + mb2_ref[...])


def _row(x):
    return x.reshape(1, -1)


@jax.jit
def kernel(atom_embedded, edge_length_embedded, edge_vec, edge_index,
           aw1, ab1, ag, abt, aw2, ab2,
           iw1, ib1, ig, ibt, iw2, ib2,
           mw1, mb1, mg, mbt, mw2, mb2):
    n, a = atom_embedded.shape
    e, f = edge_length_embedded.shape
    dn = iw2.shape[1]
    de = mw2.shape[1]
    h_inv = iw1.shape[1]
    waug = 128

    src = edge_index[0].astype(jnp.int32)
    dst = edge_index[1].astype(jnp.int32)
    zi = jnp.zeros((1,), jnp.int32)
    srcs = jnp.concatenate([src[1:], zi])
    dsts = jnp.concatenate([dst[1:], zi])

    w_src = iw1[0:a]
    w_dst = iw1[a:2 * a]
    w_e = iw1[2 * a:2 * a + f]
    w_dss = iw1[2 * a + f:3 * a + f]
    w_es = iw1[3 * a + f:3 * a + 2 * f]
    w_n = iw1[3 * a + 2 * f:]

    ncol = 128
    evs_full = jnp.concatenate([edge_vec[1:], jnp.zeros((1, 3), F32)], axis=0)
    v2d = [edge_vec[:, j].reshape(-1, ncol) for j in range(3)]
    vs2d = [evs_full[:, j].reshape(-1, ncol) for j in range(3)]
    mask2d, o3d = _make_cond(
        src.reshape(-1, ncol), srcs.reshape(-1, ncol),
        dst.reshape(-1, ncol), dsts.reshape(-1, ncol), v2d, vs2d,
        _row(aw1), _row(ab1), _row(ag), _row(abt))
    mask_e = mask2d.reshape(e, 1)
    angle16 = o3d.reshape(16, e).T

    zcol = jnp.zeros((a, 128 - h_inv), F32)
    p_tbl, q_tbl, r_tbl = _make_tables(
        atom_embedded,
        jnp.concatenate([w_src, zcol], axis=1),
        jnp.concatenate([w_dst, zcol], axis=1),
        jnp.concatenate([w_dss, zcol], axis=1), 2000)
    z_edge = _make_gather_sum(128, e, 40)(p_tbl, q_tbl, r_tbl,
                                          src, dst, dsts)

    ele = edge_length_embedded
    eles = jnp.concatenate([ele[1:], jnp.zeros((1, f), F32)], axis=0)

    w_an = aw2 @ w_n
    ib1_eff = _row(ib1) + _row(ab2) @ w_n

    be = 3200
    h2aug = pl.pallas_call(
        _edge_body,
        grid=(e // be,),
        in_specs=[
            pl.BlockSpec((be, 128), lambda i: (i, 0)),
            pl.BlockSpec((be, f), lambda i: (i, 0)),
            pl.BlockSpec((be, f), lambda i: (i, 0)),
            pl.BlockSpec((be, 1), lambda i: (i, 0)),
            pl.BlockSpec((be, f), lambda i: (i, 0)),
            pl.BlockSpec((f, h_inv), lambda i: (0, 0)),
            pl.BlockSpec((f, h_inv), lambda i: (0, 0)),
            pl.BlockSpec((f, h_inv), lambda i: (0, 0)),
            pl.BlockSpec((1, h_inv), lambda i: (0, 0)),
            pl.BlockSpec((1, h_inv), lambda i: (0, 0)),
            pl.BlockSpec((1, h_inv), lambda i: (0, 0)),
        ],
        out_specs=pl.BlockSpec((be, waug), lambda i: (i, 0)),
        out_shape=jax.ShapeDtypeStruct((e, waug), F32),
    )(z_edge, ele, eles, mask_e, angle16,
      w_e, w_es, w_an,
      ib1_eff, _row(ig), _row(ibt))

    npad = 10240
    zeros_init = jnp.zeros((npad, waug), F32)
    partials = _make_scatter_add(npad, waug, e, 40)(h2aug, dst, zeros_init)
    pa, pb = partials[0], partials[1]

    h_mrg = mw1.shape[1]
    m_s = mw1[0:dn]
    m_d = mw1[dn:2 * dn]
    m_e = mw1[2 * dn:]

    bn2 = 2048
    inv_pad, st_tbl = pl.pallas_call(
        _node_body,
        grid=(npad // bn2,),
        in_specs=[
            pl.BlockSpec((bn2, waug), lambda i: (i, 0)),
            pl.BlockSpec((bn2, waug), lambda i: (i, 0)),
            pl.BlockSpec((h_inv, dn), lambda i: (0, 0)),
            pl.BlockSpec((1, dn), lambda i: (0, 0)),
            pl.BlockSpec((dn, h_mrg), lambda i: (0, 0)),
            pl.BlockSpec((dn, h_mrg), lambda i: (0, 0)),
        ],
        out_specs=[
            pl.BlockSpec((bn2, dn), lambda i: (i, 0)),
            pl.BlockSpec((bn2, 128), lambda i: (i, 0)),
        ],
        out_shape=[
            jax.ShapeDtypeStruct((npad, dn), F32),
            jax.ShapeDtypeStruct((npad, 128), F32),
        ],
    )(pa, pb, iw2, _row(ib2), m_s, m_d)
    inv_node = inv_pad[:n]

    z5 = _make_gather_pair(h_mrg, e, 40)(st_tbl, st_tbl, src, dst)

    inv_fea_edge = pl.pallas_call(
        _merge_body,
        grid=(e // be,),
        in_specs=[
            pl.BlockSpec((be, 128), lambda i: (i, 0)),
            pl.BlockSpec((be, f), lambda i: (i, 0)),
            pl.BlockSpec((f, h_mrg), lambda i: (0, 0)),
            pl.BlockSpec((1, h_mrg), lambda i: (0, 0)),
            pl.BlockSpec((1, h_mrg), lambda i: (0, 0)),
            pl.BlockSpec((1, h_mrg), lambda i: (0, 0)),
            pl.BlockSpec((h_mrg, de), lambda i: (0, 0)),
            pl.BlockSpec((1, de), lambda i: (0, 0)),
        ],
        out_specs=pl.BlockSpec((be, de), lambda i: (i, 0)),
        out_shape=jax.ShapeDtypeStruct((e, de), F32),
    )(z5, ele, m_e,
      _row(mb1), _row(mg), _row(mbt), mw2, _row(mb2))

    return (inv_node, inv_fea_edge)


# speedup vs baseline: 1.0230x; 1.0230x over previous
import functools

import jax
import jax.numpy as jnp
from jax import lax
from jax.experimental import pallas as pl
from jax.experimental.pallas import tpu as pltpu
from jax.experimental.pallas import tpu_sc as plsc

F32 = jnp.float32

NC = 2
NS = 16
NW = NC * NS


def _cond_body(src_ref, srcs_ref, dst_ref, dsts_ref,
               vx_ref, vy_ref, vz_ref, vxs_ref, vys_ref, vzs_ref,
               aw1_ref, ab1_ref, ag_ref, abt_ref,
               mask_ref, o_ref):
    s = src_ref[...]
    ss = srcs_ref[...]
    d = dst_ref[...]
    ds = dsts_ref[...]
    cond = (s == ss) & (s != d) & (ss != ds)
    rows, cols = cond.shape
    idx = (lax.broadcasted_iota(jnp.int32, (rows, cols), 0) * cols
           + lax.broadcasted_iota(jnp.int32, (rows, cols), 1))
    last = jnp.max(jnp.where(cond, idx, -1))
    m = (cond & (idx != last)).astype(F32)
    mask_ref[...] = m
    col0 = m[:, 0:1]
    col0_up = jnp.concatenate([col0[1:], jnp.zeros((1, 1), F32)], axis=0)
    mn = jnp.concatenate([m[:, 1:], col0_up], axis=1)

    vx = vx_ref[...]
    vy = vy_ref[...]
    vz = vz_ref[...]
    vxs = vxs_ref[...]
    vys = vys_ref[...]
    vzs = vzs_ref[...]
    nv = jnp.sqrt(vx * vx + vy * vy + vz * vz)
    nvs = jnp.sqrt(vxs * vxs + vys * vys + vzs * vzs)
    dot = vx * vxs + vy * vys + vz * vzs
    cos = (m * mn * dot
           / (jnp.maximum(nv, 1e-12) * jnp.maximum(nvs, 1e-12)))

    nf = 16
    acc = jnp.zeros_like(cos)
    for j in range(nf):
        t = cos * aw1_ref[0, j] + ab1_ref[0, j]
        o_ref[j] = t
        acc = acc + t
    mu = acc * (1.0 / nf)
    acc = jnp.zeros_like(cos)
    for j in range(nf):
        dj = o_ref[j] - mu
        acc = acc + dj * dj
    inv = lax.rsqrt(acc * (1.0 / nf) + 1e-6)
    for j in range(nf):
        u = (o_ref[j] - mu) * inv * ag_ref[0, j] + abt_ref[0, j]
        o_ref[j] = u * jax.nn.sigmoid(u)


def _make_cond(src2d, srcs2d, dst2d, dsts2d, v2d, vs2d, aw1, ab1, ag, abt):
    nrow, ncol = src2d.shape
    return pl.pallas_call(
        _cond_body,
        grid=(1,),
        in_specs=([pl.BlockSpec((nrow, ncol), lambda i: (0, 0))] * 10
                  + [pl.BlockSpec(memory_space=pltpu.SMEM)] * 4),
        out_specs=[
            pl.BlockSpec((nrow, ncol), lambda i: (0, 0)),
            pl.BlockSpec((16, nrow, ncol), lambda i: (0, 0, 0)),
        ],
        out_shape=[
            jax.ShapeDtypeStruct((nrow, ncol), F32),
            jax.ShapeDtypeStruct((16, nrow, ncol), F32),
        ],
    )(src2d, srcs2d, dst2d, dsts2d, *v2d, *vs2d, aw1, ab1, ag, abt)


def _tables_body(atom_ref, wp_ref, wq_ref, wr_ref, p_ref, q_ref, r_ref):
    at = atom_ref[...]
    p_ref[...] = jnp.dot(at, wp_ref[...], preferred_element_type=F32)
    q_ref[...] = jnp.dot(at, wq_ref[...], preferred_element_type=F32)
    r_ref[...] = jnp.dot(at, wr_ref[...], preferred_element_type=F32)


def _make_tables(atom, wp, wq, wr, bn):
    n, a = atom.shape
    w = wp.shape[1]
    return pl.pallas_call(
        _tables_body,
        grid=(n // bn,),
        in_specs=[
            pl.BlockSpec((bn, a), lambda i: (i, 0)),
            pl.BlockSpec((a, w), lambda i: (0, 0)),
            pl.BlockSpec((a, w), lambda i: (0, 0)),
            pl.BlockSpec((a, w), lambda i: (0, 0)),
        ],
        out_specs=[pl.BlockSpec((bn, w), lambda i: (i, 0))] * 3,
        out_shape=[jax.ShapeDtypeStruct((n, w), F32)] * 3,
    )(atom, wp, wq, wr)


def _make_gather_sum(width, e, chunk, nbuf=2):
    per_tile = e // NW
    iters = per_tile // chunk
    mesh = plsc.VectorSubcoreMesh(core_axis_name="c", subcore_axis_name="s",
                                  num_cores=NC, num_subcores=NS)

    @functools.partial(
        pl.kernel,
        out_type=jax.ShapeDtypeStruct((e, width), F32),
        mesh=mesh,
        scratch_types=(
            [pltpu.VMEM((per_tile,), jnp.int32)] * 3
            + [pltpu.VMEM((chunk, width), F32)] * (3 * nbuf)
            + [pltpu.SemaphoreType.DMA] * (2 * nbuf)
        ),
    )
    def gsum_kernel(*refs):
        tables = refs[:3]
        idxs = refs[3:6]
        out = refs[6]
        scr = refs[7:]
        idx_all = scr[:3]
        bufs = [scr[3 + 3 * j:3 + 3 * j + 3] for j in range(nbuf)]
        gsem = scr[3 + 3 * nbuf:3 + 4 * nbuf]
        wsem = scr[3 + 4 * nbuf:3 + 5 * nbuf]
        wid = lax.axis_index("s") * NC + lax.axis_index("c")
        base0 = pl.multiple_of(wid * per_tile, 8)

        for t in range(3):
            pltpu.sync_copy(idxs[t].at[pl.ds(base0, per_tile)], idx_all[t])

        def isl(t, k):
            return idx_all[t].at[pl.ds(pl.multiple_of(k * chunk, 8), chunk)]

        def start(k, j):
            for t in range(3):
                pltpu.async_copy(tables[t].at[isl(t, k)], bufs[j][t],
                                 gsem[j])

        def wait_gathers(k, j):
            for t in range(3):
                pltpu.make_async_copy(tables[t].at[isl(t, k)], bufs[j][t],
                                      gsem[j]).wait()

        def tec_sum(j):
            b0, b1, b2 = bufs[j]

            def srow(r, _):
                for c in range(width // 16):
                    cs = pl.ds(c * 16, 16)
                    b0[r, cs] = b0[r, cs] + b1[r, cs] + b2[r, cs]
                return 0

            lax.fori_loop(0, chunk, srow, 0, unroll=False)

        def start_wb(k, j):
            b = pl.multiple_of(base0 + k * chunk, 8)
            pltpu.async_copy(bufs[j][0], out.at[pl.ds(b, chunk)], wsem[j])

        def wait_wb(k, j):
            b = pl.multiple_of(base0 + k * chunk, 8)
            pltpu.make_async_copy(bufs[j][0], out.at[pl.ds(b, chunk)],
                                  wsem[j]).wait()

        for j in range(min(nbuf, iters)):
            start(j, j)

        def body(it, _):
            for j in range(nbuf):
                k = it * nbuf + j

                @pl.when(k < iters)
                def _():
                    wait_gathers(k, j)
                    tec_sum(j)
                    start_wb(k, j)

                    @pl.when(k + nbuf < iters)
                    def _():
                        wait_wb(k, j)
                        start(k + nbuf, j)
            return 0

        lax.fori_loop(0, (iters + nbuf - 1) // nbuf, body, 0, unroll=False)
        for j in range(min(nbuf, iters)):
            klast = iters - 1 - ((iters - 1 - j) % nbuf)
            wait_wb(klast, j)

    return gsum_kernel


def _make_gather(n_tables, width, e, chunk, nbuf=4, dtype=F32):
    per_tile = e // NW
    iters = per_tile // chunk
    mesh = plsc.VectorSubcoreMesh(core_axis_name="c", subcore_axis_name="s",
                                  num_cores=NC, num_subcores=NS)

    @functools.partial(
        pl.kernel,
        out_type=[jax.ShapeDtypeStruct((e, width), dtype)] * n_tables,
        mesh=mesh,
        scratch_types=(
            [pltpu.VMEM((per_tile,), jnp.int32)] * n_tables
            + [pltpu.VMEM((chunk, width), dtype)] * nbuf
            + [pltpu.SemaphoreType.DMA] * (2 * nbuf)
        ),
    )
    def gather_kernel(*refs):
        tables = refs[:n_tables]
        idxs = refs[n_tables:2 * n_tables]
        outs = refs[2 * n_tables:3 * n_tables]
        scr = refs[3 * n_tables:]
        idx_all = scr[:n_tables]
        rows_v = scr[n_tables:n_tables + nbuf]
        gsem = scr[n_tables + nbuf:n_tables + 2 * nbuf]
        wsem = scr[n_tables + 2 * nbuf:n_tables + 3 * nbuf]
        wid = lax.axis_index("s") * NC + lax.axis_index("c")
        base0 = pl.multiple_of(wid * per_tile, 8)

        for t in range(n_tables):
            pltpu.sync_copy(idxs[t].at[pl.ds(base0, per_tile)], idx_all[t])

        for t, (tbl, out) in enumerate(zip(tables, outs)):
            iall = idx_all[t]

            def islice(k):
                return iall.at[pl.ds(pl.multiple_of(k * chunk, 8), chunk)]

            def start(k, j):
                pltpu.async_copy(tbl.at[islice(k)], rows_v[j], gsem[j])

            def wait_gather(k, j):
                pltpu.make_async_copy(tbl.at[islice(k)], rows_v[j],
                                      gsem[j]).wait()

            def start_wb(k, j):
                b = pl.multiple_of(base0 + k * chunk, 8)
                pltpu.async_copy(rows_v[j], out.at[pl.ds(b, chunk)], wsem[j])

            def wait_wb(k, j):
                b = pl.multiple_of(base0 + k * chunk, 8)
                pltpu.make_async_copy(rows_v[j], out.at[pl.ds(b, chunk)],
                                      wsem[j]).wait()

            for j in range(min(nbuf, iters)):
                start(j, j)

            def body(it, _):
                for j in range(nbuf):
                    k = it * nbuf + j

                    @pl.when(k < iters)
                    def _():
                        wait_gather(k, j)
                        start_wb(k, j)

                        @pl.when(k + nbuf < iters)
                        def _():
                            wait_wb(k, j)
                            start(k + nbuf, j)
                return 0

            lax.fori_loop(0, (iters + nbuf - 1) // nbuf, body, 0,
                          unroll=False)
            for j in range(min(nbuf, iters)):
                klast = iters - 1 - ((iters - 1 - j) % nbuf)
                wait_wb(klast, j)

    return gather_kernel


def _make_scatter_add(n, width, e, chunk):
    per_core = e // NC
    per_tile = per_core // NS
    iters = per_tile // chunk
    nrows_tile = n // NS
    mesh = plsc.VectorSubcoreMesh(core_axis_name="c", subcore_axis_name="s",
                                  num_cores=NC, num_subcores=NS)

    nbuf = 2

    @functools.partial(
        pl.kernel,
        out_type=jax.ShapeDtypeStruct((NC, n, width), F32),
        mesh=mesh,
        scratch_types=(
            [pltpu.VMEM((per_tile,), jnp.int32)]
            + [pltpu.VMEM((chunk, width), F32)] * nbuf
            + [pltpu.VMEM_SHARED((n, width), F32)]
            + [pltpu.SemaphoreType.DMA] * (2 * nbuf)
        ),
    )
    def scatter_kernel(x_hbm, dst_hbm, zeros_hbm, out_hbm,
                       idx_all, rb0, rb1, acc_sh, rs0, rs1, ss0, ss1):
        rbuf = (rb0, rb1)
        rsem = (rs0, rs1)
        ssem = (ss0, ss1)
        cid = lax.axis_index("c")
        sid = lax.axis_index("s")
        r0 = pl.multiple_of(sid * nrows_tile, 8)
        pltpu.sync_copy(zeros_hbm.at[pl.ds(r0, nrows_tile)],
                        acc_sh.at[pl.ds(r0, nrows_tile)])
        base0 = pl.multiple_of(cid * per_core + sid * per_tile, 8)
        pltpu.sync_copy(dst_hbm.at[pl.ds(base0, per_tile)], idx_all)
        plsc.subcore_barrier()

        def islice(k):
            return idx_all.at[pl.ds(pl.multiple_of(k * chunk, 8), chunk)]

        if True:
            def start_read(k, j):
                b = pl.multiple_of(base0 + k * chunk, 8)
                pltpu.async_copy(x_hbm.at[pl.ds(b, chunk)], rbuf[j], rsem[j])

            def wait_read(k, j):
                b = pl.multiple_of(base0 + k * chunk, 8)
                pltpu.make_async_copy(x_hbm.at[pl.ds(b, chunk)], rbuf[j],
                                      rsem[j]).wait()

            def start_scat(k, j):
                pltpu.async_copy(rbuf[j], acc_sh.at[islice(k)], ssem[j],
                                 add=True)

            def wait_scat(k, j):
                pltpu.make_async_copy(rbuf[j], acc_sh.at[islice(k)],
                                      ssem[j]).wait()

            for j in range(min(nbuf, iters)):
                start_read(j, j)

            def body(it, _):
                for j in range(nbuf):
                    k = it * nbuf + j

                    @pl.when(k < iters)
                    def _():
                        wait_read(k, j)
                        start_scat(k, j)

                        @pl.when(k + nbuf < iters)
                        def _():
                            wait_scat(k, j)
                            start_read(k + nbuf, j)
                return 0

            lax.fori_loop(0, (iters + nbuf - 1) // nbuf, body, 0,
                          unroll=False)
            for j in range(min(nbuf, iters)):
                klast = iters - 1 - ((iters - 1 - j) % nbuf)
                wait_scat(klast, j)

        plsc.subcore_barrier()
        pltpu.sync_copy(acc_sh.at[pl.ds(r0, nrows_tile)],
                        out_hbm.at[cid].at[pl.ds(r0, nrows_tile)])

    return scatter_kernel


def _edge_body(z_ref, ele_ref, eles_ref,
               m_ref, ang_ref,
               we_ref, wes_ref, wan_ref,
               ib1_ref, ig_ref, ibt_ref,
               out_ref):
    b = m_ref.shape[0]
    hw = we_ref.shape[1]
    m = m_ref[...]

    lin = (z_ref[...][:, :hw]
           + jnp.dot(ele_ref[...], we_ref[...], preferred_element_type=F32)
           + jnp.dot(eles_ref[...], wes_ref[...], preferred_element_type=F32)
           + jnp.dot(ang_ref[...], wan_ref[...], preferred_element_type=F32))
    h = m * lin + ib1_ref[...]
    jmat = jnp.full((hw, hw), 1.0 / hw, F32)
    mu = jnp.dot(h, jmat, preferred_element_type=F32)
    s2 = jnp.dot(h * h, jmat, preferred_element_type=F32)
    var = jnp.maximum(s2 - mu * mu, 0.0)
    h = (h - mu) * lax.rsqrt(var + 1e-6) * ig_ref[...] + ibt_ref[...]
    h = h * jax.nn.sigmoid(h)
    out_ref[...] = jnp.concatenate(
        [h, jnp.ones((b, 1), F32), jnp.zeros((b, 127 - hw), F32)], axis=1)


def _node_body(pa_ref, pb_ref, iw2_ref, ib2_ref, inv_ref):
    hw = iw2_ref.shape[0]
    acc = pa_ref[...] + pb_ref[...]
    h2s = acc[:, :hw]
    cnt = acc[:, hw:hw + 1]
    inv_ref[...] = (jnp.dot(h2s, iw2_ref[...], preferred_element_type=F32)
                    + cnt * ib2_ref[...])


def _merge_body(g4_ref, g5_ref, ele_ref, ms_ref, md_ref, me_ref,
                mb1_ref, mg_ref, mbt_ref, mw2_ref, mb2_ref, out_ref):
    h = (jnp.dot(g4_ref[...], ms_ref[...], preferred_element_type=F32)
         + jnp.dot(g5_ref[...], md_ref[...], preferred_element_type=F32)
         + jnp.dot(ele_ref[...], me_ref[...], preferred_element_type=F32)
         + mb1_ref[...])
    hm = ms_ref.shape[1]
    jmat = jnp.full((hm, hm), 1.0 / hm, F32)
    mu = jnp.dot(h, jmat, preferred_element_type=F32)
    s2 = jnp.dot(h * h, jmat, preferred_element_type=F32)
    var = jnp.maximum(s2 - mu * mu, 0.0)
    h = (h - mu) * lax.rsqrt(var + 1e-6) * mg_ref[...] + mbt_ref[...]
    h = h * jax.nn.sigmoid(h)
    out_ref[...] = (jnp.dot(h, mw2_ref[...], preferred_element_type=F32)
                    + mb2_ref[...])


def _row(x):
    return x.reshape(1, -1)


@jax.jit
def kernel(atom_embedded, edge_length_embedded, edge_vec, edge_index,
           aw1, ab1, ag, abt, aw2, ab2,
           iw1, ib1, ig, ibt, iw2, ib2,
           mw1, mb1, mg, mbt, mw2, mb2):
    n, a = atom_embedded.shape
    e, f = edge_length_embedded.shape
    dn = iw2.shape[1]
    de = mw2.shape[1]
    h_inv = iw1.shape[1]
    waug = 128

    src = edge_index[0].astype(jnp.int32)
    dst = edge_index[1].astype(jnp.int32)
    zi = jnp.zeros((1,), jnp.int32)
    srcs = jnp.concatenate([src[1:], zi])
    dsts = jnp.concatenate([dst[1:], zi])

    w_src = iw1[0:a]
    w_dst = iw1[a:2 * a]
    w_e = iw1[2 * a:2 * a + f]
    w_dss = iw1[2 * a + f:3 * a + f]
    w_es = iw1[3 * a + f:3 * a + 2 * f]
    w_n = iw1[3 * a + 2 * f:]

    ncol = 128
    evs_full = jnp.concatenate([edge_vec[1:], jnp.zeros((1, 3), F32)], axis=0)
    v2d = [edge_vec[:, j].reshape(-1, ncol) for j in range(3)]
    vs2d = [evs_full[:, j].reshape(-1, ncol) for j in range(3)]
    mask2d, o3d = _make_cond(
        src.reshape(-1, ncol), srcs.reshape(-1, ncol),
        dst.reshape(-1, ncol), dsts.reshape(-1, ncol), v2d, vs2d,
        _row(aw1), _row(ab1), _row(ag), _row(abt))
    mask_e = mask2d.reshape(e, 1)
    angle16 = o3d.reshape(16, e).T

    zcol = jnp.zeros((a, 128 - h_inv), F32)
    p_tbl, q_tbl, r_tbl = _make_tables(
        atom_embedded,
        jnp.concatenate([w_src, zcol], axis=1),
        jnp.concatenate([w_dst, zcol], axis=1),
        jnp.concatenate([w_dss, zcol], axis=1), 2000)
    z_edge = _make_gather_sum(128, e, 40)(p_tbl, q_tbl, r_tbl,
                                          src, dst, dsts)

    ele = edge_length_embedded
    eles = jnp.concatenate([ele[1:], jnp.zeros((1, f), F32)], axis=0)

    w_an = aw2 @ w_n
    ib1_eff = _row(ib1) + _row(ab2) @ w_n

    be = 3200
    h2aug = pl.pallas_call(
        _edge_body,
        grid=(e // be,),
        in_specs=[
            pl.BlockSpec((be, 128), lambda i: (i, 0)),
            pl.BlockSpec((be, f), lambda i: (i, 0)),
            pl.BlockSpec((be, f), lambda i: (i, 0)),
            pl.BlockSpec((be, 1), lambda i: (i, 0)),
            pl.BlockSpec((be, f), lambda i: (i, 0)),
            pl.BlockSpec((f, h_inv), lambda i: (0, 0)),
            pl.BlockSpec((f, h_inv), lambda i: (0, 0)),
            pl.BlockSpec((f, h_inv), lambda i: (0, 0)),
            pl.BlockSpec((1, h_inv), lambda i: (0, 0)),
            pl.BlockSpec((1, h_inv), lambda i: (0, 0)),
            pl.BlockSpec((1, h_inv), lambda i: (0, 0)),
        ],
        out_specs=pl.BlockSpec((be, waug), lambda i: (i, 0)),
        out_shape=jax.ShapeDtypeStruct((e, waug), F32),
    )(z_edge, ele, eles, mask_e, angle16,
      w_e, w_es, w_an,
      ib1_eff, _row(ig), _row(ibt))

    npad = 10240
    zeros_init = jnp.zeros((npad, waug), F32)
    partials = _make_scatter_add(npad, waug, e, 40)(h2aug, dst, zeros_init)
    pa, pb = partials[0], partials[1]

    bn2 = 2048
    inv_pad = pl.pallas_call(
        _node_body,
        grid=(npad // bn2,),
        in_specs=[
            pl.BlockSpec((bn2, waug), lambda i: (i, 0)),
            pl.BlockSpec((bn2, waug), lambda i: (i, 0)),
            pl.BlockSpec((h_inv, dn), lambda i: (0, 0)),
            pl.BlockSpec((1, dn), lambda i: (0, 0)),
        ],
        out_specs=pl.BlockSpec((bn2, dn), lambda i: (i, 0)),
        out_shape=jax.ShapeDtypeStruct((npad, dn), F32),
    )(pa, pb, iw2, _row(ib2))
    inv_node = inv_pad[:n]

    g4, g5 = _make_gather(2, dn, e, 200)(inv_pad, inv_pad, src, dst)

    h_mrg = mw1.shape[1]
    m_s = mw1[0:dn]
    m_d = mw1[dn:2 * dn]
    m_e = mw1[2 * dn:]
    inv_fea_edge = pl.pallas_call(
        _merge_body,
        grid=(e // be,),
        in_specs=[
            pl.BlockSpec((be, dn), lambda i: (i, 0)),
            pl.BlockSpec((be, dn), lambda i: (i, 0)),
            pl.BlockSpec((be, f), lambda i: (i, 0)),
            pl.BlockSpec((dn, h_mrg), lambda i: (0, 0)),
            pl.BlockSpec((dn, h_mrg), lambda i: (0, 0)),
            pl.BlockSpec((f, h_mrg), lambda i: (0, 0)),
            pl.BlockSpec((1, h_mrg), lambda i: (0, 0)),
            pl.BlockSpec((1, h_mrg), lambda i: (0, 0)),
            pl.BlockSpec((1, h_mrg), lambda i: (0, 0)),
            pl.BlockSpec((h_mrg, de), lambda i: (0, 0)),
            pl.BlockSpec((1, de), lambda i: (0, 0)),
        ],
        out_specs=pl.BlockSpec((be, de), lambda i: (i, 0)),
        out_shape=jax.ShapeDtypeStruct((e, de), F32),
    )(g4, g5, ele, m_s, m_d, m_e,
      _row(mb1), _row(mg), _row(mbt), mw2, _row(mb2))

    return (inv_node, inv_fea_edge)


# speedup vs baseline: 1.0674x; 1.0434x over previous
import functools

import jax
import jax.numpy as jnp
from jax import lax
from jax.experimental import pallas as pl
from jax.experimental.pallas import tpu as pltpu
from jax.experimental.pallas import tpu_sc as plsc

F32 = jnp.float32

NC = 2
NS = 16
NW = NC * NS


def _cond_body(src_ref, srcs_ref, dst_ref, dsts_ref,
               vx_ref, vy_ref, vz_ref, vxs_ref, vys_ref, vzs_ref,
               aw1_ref, ab1_ref, ag_ref, abt_ref,
               mask_ref, o_ref):
    s = src_ref[...]
    ss = srcs_ref[...]
    d = dst_ref[...]
    ds = dsts_ref[...]
    cond = (s == ss) & (s != d) & (ss != ds)
    rows, cols = cond.shape
    idx = (lax.broadcasted_iota(jnp.int32, (rows, cols), 0) * cols
           + lax.broadcasted_iota(jnp.int32, (rows, cols), 1))
    last = jnp.max(jnp.where(cond, idx, -1))
    m = (cond & (idx != last)).astype(F32)
    mask_ref[...] = m
    col0 = m[:, 0:1]
    col0_up = jnp.concatenate([col0[1:], jnp.zeros((1, 1), F32)], axis=0)
    mn = jnp.concatenate([m[:, 1:], col0_up], axis=1)

    vx = vx_ref[...]
    vy = vy_ref[...]
    vz = vz_ref[...]
    vxs = vxs_ref[...]
    vys = vys_ref[...]
    vzs = vzs_ref[...]
    nv = jnp.sqrt(vx * vx + vy * vy + vz * vz)
    nvs = jnp.sqrt(vxs * vxs + vys * vys + vzs * vzs)
    dot = vx * vxs + vy * vys + vz * vzs
    cos = (m * mn * dot
           / (jnp.maximum(nv, 1e-12) * jnp.maximum(nvs, 1e-12)))

    nf = 16
    acc = jnp.zeros_like(cos)
    for j in range(nf):
        t = cos * aw1_ref[0, j] + ab1_ref[0, j]
        o_ref[j] = t
        acc = acc + t
    mu = acc * (1.0 / nf)
    acc = jnp.zeros_like(cos)
    for j in range(nf):
        dj = o_ref[j] - mu
        acc = acc + dj * dj
    inv = lax.rsqrt(acc * (1.0 / nf) + 1e-6)
    for j in range(nf):
        u = (o_ref[j] - mu) * inv * ag_ref[0, j] + abt_ref[0, j]
        o_ref[j] = u * jax.nn.sigmoid(u)


def _make_cond(src2d, srcs2d, dst2d, dsts2d, v2d, vs2d, aw1, ab1, ag, abt):
    nrow, ncol = src2d.shape
    return pl.pallas_call(
        _cond_body,
        grid=(1,),
        in_specs=([pl.BlockSpec((nrow, ncol), lambda i: (0, 0))] * 10
                  + [pl.BlockSpec(memory_space=pltpu.SMEM)] * 4),
        out_specs=[
            pl.BlockSpec((nrow, ncol), lambda i: (0, 0)),
            pl.BlockSpec((16, nrow, ncol), lambda i: (0, 0, 0)),
        ],
        out_shape=[
            jax.ShapeDtypeStruct((nrow, ncol), F32),
            jax.ShapeDtypeStruct((16, nrow, ncol), F32),
        ],
    )(src2d, srcs2d, dst2d, dsts2d, *v2d, *vs2d, aw1, ab1, ag, abt)


def _tables_body(atom_ref, wp_ref, wq_ref, wr_ref, p_ref, q_ref, r_ref):
    at = atom_ref[...]
    p_ref[...] = jnp.dot(at, wp_ref[...], preferred_element_type=F32)
    q_ref[...] = jnp.dot(at, wq_ref[...], preferred_element_type=F32)
    r_ref[...] = jnp.dot(at, wr_ref[...], preferred_element_type=F32)


def _make_tables(atom, wp, wq, wr, bn):
    n, a = atom.shape
    w = wp.shape[1]
    return pl.pallas_call(
        _tables_body,
        grid=(n // bn,),
        in_specs=[
            pl.BlockSpec((bn, a), lambda i: (i, 0)),
            pl.BlockSpec((a, w), lambda i: (0, 0)),
            pl.BlockSpec((a, w), lambda i: (0, 0)),
            pl.BlockSpec((a, w), lambda i: (0, 0)),
        ],
        out_specs=[pl.BlockSpec((bn, w), lambda i: (i, 0))] * 3,
        out_shape=[jax.ShapeDtypeStruct((n, w), F32)] * 3,
    )(atom, wp, wq, wr)


def _make_gather_sum(width, e, chunk, nbuf=2):
    per_tile = e // NW
    iters = per_tile // chunk
    mesh = plsc.VectorSubcoreMesh(core_axis_name="c", subcore_axis_name="s",
                                  num_cores=NC, num_subcores=NS)

    @functools.partial(
        pl.kernel,
        out_type=jax.ShapeDtypeStruct((e, width), F32),
        mesh=mesh,
        scratch_types=(
            [pltpu.VMEM((per_tile,), jnp.int32)] * 3
            + [pltpu.VMEM((chunk, width), F32)] * (3 * nbuf)
            + [pltpu.SemaphoreType.DMA] * (2 * nbuf)
        ),
    )
    def gsum_kernel(*refs):
        tables = refs[:3]
        idxs = refs[3:6]
        out = refs[6]
        scr = refs[7:]
        idx_all = scr[:3]
        bufs = [scr[3 + 3 * j:3 + 3 * j + 3] for j in range(nbuf)]
        gsem = scr[3 + 3 * nbuf:3 + 4 * nbuf]
        wsem = scr[3 + 4 * nbuf:3 + 5 * nbuf]
        wid = lax.axis_index("s") * NC + lax.axis_index("c")
        base0 = pl.multiple_of(wid * per_tile, 8)

        for t in range(3):
            pltpu.sync_copy(idxs[t].at[pl.ds(base0, per_tile)], idx_all[t])

        def isl(t, k):
            return idx_all[t].at[pl.ds(pl.multiple_of(k * chunk, 8), chunk)]

        def start(k, j):
            for t in range(3):
                pltpu.async_copy(tables[t].at[isl(t, k)], bufs[j][t],
                                 gsem[j])

        def wait_gathers(k, j):
            for t in range(3):
                pltpu.make_async_copy(tables[t].at[isl(t, k)], bufs[j][t],
                                      gsem[j]).wait()

        def tec_sum(j):
            b0, b1, b2 = bufs[j]

            def srow(r, _):
                for c in range(width // 16):
                    cs = pl.ds(c * 16, 16)
                    b0[r, cs] = b0[r, cs] + b1[r, cs] + b2[r, cs]
                return 0

            lax.fori_loop(0, chunk, srow, 0, unroll=False)

        def start_wb(k, j):
            b = pl.multiple_of(base0 + k * chunk, 8)
            pltpu.async_copy(bufs[j][0], out.at[pl.ds(b, chunk)], wsem[j])

        def wait_wb(k, j):
            b = pl.multiple_of(base0 + k * chunk, 8)
            pltpu.make_async_copy(bufs[j][0], out.at[pl.ds(b, chunk)],
                                  wsem[j]).wait()

        for j in range(min(nbuf, iters)):
            start(j, j)

        def body(it, _):
            for j in range(nbuf):
                k = it * nbuf + j

                @pl.when(k < iters)
                def _():
                    wait_gathers(k, j)
                    tec_sum(j)
                    start_wb(k, j)

                    @pl.when(k + nbuf < iters)
                    def _():
                        wait_wb(k, j)
                        start(k + nbuf, j)
            return 0

        lax.fori_loop(0, (iters + nbuf - 1) // nbuf, body, 0, unroll=False)
        for j in range(min(nbuf, iters)):
            klast = iters - 1 - ((iters - 1 - j) % nbuf)
            wait_wb(klast, j)

    return gsum_kernel


def _make_gather_pair(hm, e, chunk, nbuf=2):
    width = 128
    per_tile = e // NW
    iters = per_tile // chunk
    mesh = plsc.VectorSubcoreMesh(core_axis_name="c", subcore_axis_name="s",
                                  num_cores=NC, num_subcores=NS)

    @functools.partial(
        pl.kernel,
        out_type=jax.ShapeDtypeStruct((e, width), F32),
        mesh=mesh,
        scratch_types=(
            [pltpu.VMEM((per_tile,), jnp.int32)] * 2
            + [pltpu.VMEM((chunk, width), F32)] * (2 * nbuf)
            + [pltpu.SemaphoreType.DMA] * (2 * nbuf)
        ),
    )
    def gpair_kernel(*refs):
        tables = refs[:2]
        idxs = refs[2:4]
        out = refs[4]
        scr = refs[5:]
        idx_all = scr[:2]
        bufs = [scr[2 + 2 * j:2 + 2 * j + 2] for j in range(nbuf)]
        gsem = scr[2 + 2 * nbuf:2 + 3 * nbuf]
        wsem = scr[2 + 3 * nbuf:2 + 4 * nbuf]
        wid = lax.axis_index("s") * NC + lax.axis_index("c")
        base0 = pl.multiple_of(wid * per_tile, 8)

        for t in range(2):
            pltpu.sync_copy(idxs[t].at[pl.ds(base0, per_tile)], idx_all[t])

        def isl(t, k):
            return idx_all[t].at[pl.ds(pl.multiple_of(k * chunk, 8), chunk)]

        def start(k, j):
            for t in range(2):
                pltpu.async_copy(tables[t].at[isl(t, k)], bufs[j][t],
                                 gsem[j])

        def wait_gathers(k, j):
            for t in range(2):
                pltpu.make_async_copy(tables[t].at[isl(t, k)], bufs[j][t],
                                      gsem[j]).wait()

        def tec_pair(j):
            b0, b1 = bufs[j]

            def srow(r, _):
                for c in range(hm // 16):
                    cs = pl.ds(hm + c * 16, 16)
                    b0[r, cs] = b1[r, cs]
                return 0

            lax.fori_loop(0, chunk, srow, 0, unroll=False)

        def start_wb(k, j):
            b = pl.multiple_of(base0 + k * chunk, 8)
            pltpu.async_copy(bufs[j][0], out.at[pl.ds(b, chunk)], wsem[j])

        def wait_wb(k, j):
            b = pl.multiple_of(base0 + k * chunk, 8)
            pltpu.make_async_copy(bufs[j][0], out.at[pl.ds(b, chunk)],
                                  wsem[j]).wait()

        for j in range(min(nbuf, iters)):
            start(j, j)

        def body(it, _):
            for j in range(nbuf):
                k = it * nbuf + j

                @pl.when(k < iters)
                def _():
                    wait_gathers(k, j)
                    tec_pair(j)
                    start_wb(k, j)

                    @pl.when(k + nbuf < iters)
                    def _():
                        wait_wb(k, j)
                        start(k + nbuf, j)
            return 0

        lax.fori_loop(0, (iters + nbuf - 1) // nbuf, body, 0, unroll=False)
        for j in range(min(nbuf, iters)):
            klast = iters - 1 - ((iters - 1 - j) % nbuf)
            wait_wb(klast, j)

    return gpair_kernel


def _make_gather(n_tables, width, e, chunk, nbuf=4, dtype=F32):
    per_tile = e // NW
    iters = per_tile // chunk
    mesh = plsc.VectorSubcoreMesh(core_axis_name="c", subcore_axis_name="s",
                                  num_cores=NC, num_subcores=NS)

    @functools.partial(
        pl.kernel,
        out_type=[jax.ShapeDtypeStruct((e, width), dtype)] * n_tables,
        mesh=mesh,
        scratch_types=(
            [pltpu.VMEM((per_tile,), jnp.int32)] * n_tables
            + [pltpu.VMEM((chunk, width), dtype)] * nbuf
            + [pltpu.SemaphoreType.DMA] * (2 * nbuf)
        ),
    )
    def gather_kernel(*refs):
        tables = refs[:n_tables]
        idxs = refs[n_tables:2 * n_tables]
        outs = refs[2 * n_tables:3 * n_tables]
        scr = refs[3 * n_tables:]
        idx_all = scr[:n_tables]
        rows_v = scr[n_tables:n_tables + nbuf]
        gsem = scr[n_tables + nbuf:n_tables + 2 * nbuf]
        wsem = scr[n_tables + 2 * nbuf:n_tables + 3 * nbuf]
        wid = lax.axis_index("s") * NC + lax.axis_index("c")
        base0 = pl.multiple_of(wid * per_tile, 8)

        for t in range(n_tables):
            pltpu.sync_copy(idxs[t].at[pl.ds(base0, per_tile)], idx_all[t])

        for t, (tbl, out) in enumerate(zip(tables, outs)):
            iall = idx_all[t]

            def islice(k):
                return iall.at[pl.ds(pl.multiple_of(k * chunk, 8), chunk)]

            def start(k, j):
                pltpu.async_copy(tbl.at[islice(k)], rows_v[j], gsem[j])

            def wait_gather(k, j):
                pltpu.make_async_copy(tbl.at[islice(k)], rows_v[j],
                                      gsem[j]).wait()

            def start_wb(k, j):
                b = pl.multiple_of(base0 + k * chunk, 8)
                pltpu.async_copy(rows_v[j], out.at[pl.ds(b, chunk)], wsem[j])

            def wait_wb(k, j):
                b = pl.multiple_of(base0 + k * chunk, 8)
                pltpu.make_async_copy(rows_v[j], out.at[pl.ds(b, chunk)],
                                      wsem[j]).wait()

            for j in range(min(nbuf, iters)):
                start(j, j)

            def body(it, _):
                for j in range(nbuf):
                    k = it * nbuf + j

                    @pl.when(k < iters)
                    def _():
                        wait_gather(k, j)
                        start_wb(k, j)

                        @pl.when(k + nbuf < iters)
                        def _():
                            wait_wb(k, j)
                            start(k + nbuf, j)
                return 0

            lax.fori_loop(0, (iters + nbuf - 1) // nbuf, body, 0,
                          unroll=False)
            for j in range(min(nbuf, iters)):
                klast = iters - 1 - ((iters - 1 - j) % nbuf)
                wait_wb(klast, j)

    return gather_kernel


def _make_scatter_add(n, width, e, chunk):
    per_core = e // NC
    per_tile = per_core // NS
    iters = per_tile // chunk
    nrows_tile = n // NS
    mesh = plsc.VectorSubcoreMesh(core_axis_name="c", subcore_axis_name="s",
                                  num_cores=NC, num_subcores=NS)

    nbuf = 2

    @functools.partial(
        pl.kernel,
        out_type=jax.ShapeDtypeStruct((NC, n, width), F32),
        mesh=mesh,
        scratch_types=(
            [pltpu.VMEM((per_tile,), jnp.int32)]
            + [pltpu.VMEM((chunk, width), F32)] * nbuf
            + [pltpu.VMEM_SHARED((n, width), F32)]
            + [pltpu.SemaphoreType.DMA] * (2 * nbuf)
        ),
    )
    def scatter_kernel(x_hbm, dst_hbm, zeros_hbm, out_hbm,
                       idx_all, rb0, rb1, acc_sh, rs0, rs1, ss0, ss1):
        rbuf = (rb0, rb1)
        rsem = (rs0, rs1)
        ssem = (ss0, ss1)
        cid = lax.axis_index("c")
        sid = lax.axis_index("s")
        r0 = pl.multiple_of(sid * nrows_tile, 8)
        pltpu.sync_copy(zeros_hbm.at[pl.ds(r0, nrows_tile)],
                        acc_sh.at[pl.ds(r0, nrows_tile)])
        base0 = pl.multiple_of(cid * per_core + sid * per_tile, 8)
        pltpu.sync_copy(dst_hbm.at[pl.ds(base0, per_tile)], idx_all)
        plsc.subcore_barrier()

        def islice(k):
            return idx_all.at[pl.ds(pl.multiple_of(k * chunk, 8), chunk)]

        if True:
            def start_read(k, j):
                b = pl.multiple_of(base0 + k * chunk, 8)
                pltpu.async_copy(x_hbm.at[pl.ds(b, chunk)], rbuf[j], rsem[j])

            def wait_read(k, j):
                b = pl.multiple_of(base0 + k * chunk, 8)
                pltpu.make_async_copy(x_hbm.at[pl.ds(b, chunk)], rbuf[j],
                                      rsem[j]).wait()

            def start_scat(k, j):
                pltpu.async_copy(rbuf[j], acc_sh.at[islice(k)], ssem[j],
                                 add=True)

            def wait_scat(k, j):
                pltpu.make_async_copy(rbuf[j], acc_sh.at[islice(k)],
                                      ssem[j]).wait()

            for j in range(min(nbuf, iters)):
                start_read(j, j)

            def body(it, _):
                for j in range(nbuf):
                    k = it * nbuf + j

                    @pl.when(k < iters)
                    def _():
                        wait_read(k, j)
                        start_scat(k, j)

                        @pl.when(k + nbuf < iters)
                        def _():
                            wait_scat(k, j)
                            start_read(k + nbuf, j)
                return 0

            lax.fori_loop(0, (iters + nbuf - 1) // nbuf, body, 0,
                          unroll=False)
            for j in range(min(nbuf, iters)):
                klast = iters - 1 - ((iters - 1 - j) % nbuf)
                wait_scat(klast, j)

        plsc.subcore_barrier()
        pltpu.sync_copy(acc_sh.at[pl.ds(r0, nrows_tile)],
                        out_hbm.at[cid].at[pl.ds(r0, nrows_tile)])

    return scatter_kernel


def _edge_body(z_ref, ele_ref, eles_ref,
               m_ref, ang_ref,
               we_ref, wes_ref, wan_ref,
               ib1_ref, ig_ref, ibt_ref,
               out_ref):
    b = m_ref.shape[0]
    hw = we_ref.shape[1]
    m = m_ref[...]

    lin = (z_ref[...][:, :hw]
           + jnp.dot(ele_ref[...], we_ref[...], preferred_element_type=F32)
           + jnp.dot(eles_ref[...], wes_ref[...], preferred_element_type=F32)
           + jnp.dot(ang_ref[...], wan_ref[...], preferred_element_type=F32))
    h = m * lin + ib1_ref[...]
    jmat = jnp.full((hw, hw), 1.0 / hw, F32)
    mu = jnp.dot(h, jmat, preferred_element_type=F32)
    s2 = jnp.dot(h * h, jmat, preferred_element_type=F32)
    var = jnp.maximum(s2 - mu * mu, 0.0)
    h = (h - mu) * lax.rsqrt(var + 1e-6) * ig_ref[...] + ibt_ref[...]
    h = h * jax.nn.sigmoid(h)
    out_ref[...] = jnp.concatenate(
        [h, jnp.ones((b, 1), F32), jnp.zeros((b, 127 - hw), F32)], axis=1)


def _node_body(pa_ref, pb_ref, iw2_ref, ib2_ref, ms_ref, md_ref,
               inv_ref, st_ref):
    hw = iw2_ref.shape[0]
    hm = ms_ref.shape[1]
    b = pa_ref.shape[0]
    acc = pa_ref[...] + pb_ref[...]
    h2s = acc[:, :hw]
    cnt = acc[:, hw:hw + 1]
    inv = (jnp.dot(h2s, iw2_ref[...], preferred_element_type=F32)
           + cnt * ib2_ref[...])
    inv_ref[...] = inv
    st_ref[...] = jnp.concatenate(
        [jnp.dot(inv, ms_ref[...], preferred_element_type=F32),
         jnp.dot(inv, md_ref[...], preferred_element_type=F32),
         jnp.zeros((b, 128 - 2 * hm), F32)], axis=1)


def _merge_body(z5_ref, ele_ref, me_ref,
                mb1_ref, mg_ref, mbt_ref, mw2_ref, mb2_ref, out_ref):
    hm = me_ref.shape[1]
    z5 = z5_ref[...]
    h = (z5[:, :hm] + z5[:, hm:2 * hm]
         + jnp.dot(ele_ref[...], me_ref[...], preferred_element_type=F32)
         + mb1_ref[...])
    jmat = jnp.full((hm, hm), 1.0 / hm, F32)
    mu = jnp.dot(h, jmat, preferred_element_type=F32)
    s2 = jnp.dot(h * h, jmat, preferred_element_type=F32)
    var = jnp.maximum(s2 - mu * mu, 0.0)
    h = (h - mu) * lax.rsqrt(var + 1e-6) * mg_ref[...] + mbt_ref[...]
    h = h * jax.nn.sigmoid(h)
    out_ref[...] = (jnp.dot(h, mw2_ref[...], preferred_element_type=F32)
                    + mb2_ref[...])


def _row(x):
    return x.reshape(1, -1)


@jax.jit
def kernel(atom_embedded, edge_length_embedded, edge_vec, edge_index,
           aw1, ab1, ag, abt, aw2, ab2,
           iw1, ib1, ig, ibt, iw2, ib2,
           mw1, mb1, mg, mbt, mw2, mb2):
    n, a = atom_embedded.shape
    e, f = edge_length_embedded.shape
    dn = iw2.shape[1]
    de = mw2.shape[1]
    h_inv = iw1.shape[1]
    waug = 128

    src = edge_index[0].astype(jnp.int32)
    dst = edge_index[1].astype(jnp.int32)
    zi = jnp.zeros((1,), jnp.int32)
    srcs = jnp.concatenate([src[1:], zi])
    dsts = jnp.concatenate([dst[1:], zi])

    w_src = iw1[0:a]
    w_dst = iw1[a:2 * a]
    w_e = iw1[2 * a:2 * a + f]
    w_dss = iw1[2 * a + f:3 * a + f]
    w_es = iw1[3 * a + f:3 * a + 2 * f]
    w_n = iw1[3 * a + 2 * f:]

    ncol = 128
    evs_full = jnp.concatenate([edge_vec[1:], jnp.zeros((1, 3), F32)], axis=0)
    v2d = [edge_vec[:, j].reshape(-1, ncol) for j in range(3)]
    vs2d = [evs_full[:, j].reshape(-1, ncol) for j in range(3)]
    mask2d, o3d = _make_cond(
        src.reshape(-1, ncol), srcs.reshape(-1, ncol),
        dst.reshape(-1, ncol), dsts.reshape(-1, ncol), v2d, vs2d,
        _row(aw1), _row(ab1), _row(ag), _row(abt))
    mask_e = mask2d.reshape(e, 1)
    angle16 = o3d.reshape(16, e).T

    zcol = jnp.zeros((a, 128 - h_inv), F32)
    p_tbl, q_tbl, r_tbl = _make_tables(
        atom_embedded,
        jnp.concatenate([w_src, zcol], axis=1),
        jnp.concatenate([w_dst, zcol], axis=1),
        jnp.concatenate([w_dss, zcol], axis=1), 2000)
    z_edge = _make_gather_sum(128, e, 40)(p_tbl, q_tbl, r_tbl,
                                          src, dst, dsts)

    ele = edge_length_embedded
    eles = jnp.concatenate([ele[1:], jnp.zeros((1, f), F32)], axis=0)

    w_an = aw2 @ w_n
    ib1_eff = _row(ib1) + _row(ab2) @ w_n

    be = 3200
    h2aug = pl.pallas_call(
        _edge_body,
        grid=(e // be,),
        in_specs=[
            pl.BlockSpec((be, 128), lambda i: (i, 0)),
            pl.BlockSpec((be, f), lambda i: (i, 0)),
            pl.BlockSpec((be, f), lambda i: (i, 0)),
            pl.BlockSpec((be, 1), lambda i: (i, 0)),
            pl.BlockSpec((be, f), lambda i: (i, 0)),
            pl.BlockSpec((f, h_inv), lambda i: (0, 0)),
            pl.BlockSpec((f, h_inv), lambda i: (0, 0)),
            pl.BlockSpec((f, h_inv), lambda i: (0, 0)),
            pl.BlockSpec((1, h_inv), lambda i: (0, 0)),
            pl.BlockSpec((1, h_inv), lambda i: (0, 0)),
            pl.BlockSpec((1, h_inv), lambda i: (0, 0)),
        ],
        out_specs=pl.BlockSpec((be, waug), lambda i: (i, 0)),
        out_shape=jax.ShapeDtypeStruct((e, waug), F32),
    )(z_edge, ele, eles, mask_e, angle16,
      w_e, w_es, w_an,
      ib1_eff, _row(ig), _row(ibt))

    npad = 10240
    zeros_init = jnp.zeros((npad, waug), F32)
    partials = _make_scatter_add(npad, waug, e, 40)(h2aug, dst, zeros_init)
    pa, pb = partials[0], partials[1]

    h_mrg = mw1.shape[1]
    m_s = mw1[0:dn]
    m_d = mw1[dn:2 * dn]
    m_e = mw1[2 * dn:]

    bn2 = 2048
    inv_pad, st_tbl = pl.pallas_call(
        _node_body,
        grid=(npad // bn2,),
        in_specs=[
            pl.BlockSpec((bn2, waug), lambda i: (i, 0)),
            pl.BlockSpec((bn2, waug), lambda i: (i, 0)),
            pl.BlockSpec((h_inv, dn), lambda i: (0, 0)),
            pl.BlockSpec((1, dn), lambda i: (0, 0)),
            pl.BlockSpec((dn, h_mrg), lambda i: (0, 0)),
            pl.BlockSpec((dn, h_mrg), lambda i: (0, 0)),
        ],
        out_specs=[
            pl.BlockSpec((bn2, dn), lambda i: (i, 0)),
            pl.BlockSpec((bn2, 128), lambda i: (i, 0)),
        ],
        out_shape=[
            jax.ShapeDtypeStruct((npad, dn), F32),
            jax.ShapeDtypeStruct((npad, 128), F32),
        ],
    )(pa, pb, iw2, _row(ib2), m_s, m_d)
    inv_node = inv_pad[:n]

    z5 = _make_gather_pair(h_mrg, e, 200)(st_tbl, st_tbl, src, dst)

    inv_fea_edge = pl.pallas_call(
        _merge_body,
        grid=(e // be,),
        in_specs=[
            pl.BlockSpec((be, 128), lambda i: (i, 0)),
            pl.BlockSpec((be, f), lambda i: (i, 0)),
            pl.BlockSpec((f, h_mrg), lambda i: (0, 0)),
            pl.BlockSpec((1, h_mrg), lambda i: (0, 0)),
            pl.BlockSpec((1, h_mrg), lambda i: (0, 0)),
            pl.BlockSpec((1, h_mrg), lambda i: (0, 0)),
            pl.BlockSpec((h_mrg, de), lambda i: (0, 0)),
            pl.BlockSpec((1, de), lambda i: (0, 0)),
        ],
        out_specs=pl.BlockSpec((be, de), lambda i: (i, 0)),
        out_shape=jax.ShapeDtypeStruct((e, de), F32),
    )(z5, ele, m_e,
      _row(mb1), _row(mg), _row(mbt), mw2, _row(mb2))

    return (inv_node, inv_fea_edge)


# speedup vs baseline: 1.0994x; 1.0300x over previous
import functools

import jax
import jax.numpy as jnp
from jax import lax
from jax.experimental import pallas as pl
from jax.experimental.pallas import tpu as pltpu
from jax.experimental.pallas import tpu_sc as plsc

F32 = jnp.float32

NC = 2
NS = 16
NW = NC * NS


def _cond_body(src_ref, srcs_ref, dst_ref, dsts_ref,
               vx_ref, vy_ref, vz_ref, vxs_ref, vys_ref, vzs_ref,
               aw1_ref, ab1_ref, ag_ref, abt_ref,
               mask_ref, o_ref):
    s = src_ref[...]
    ss = srcs_ref[...]
    d = dst_ref[...]
    ds = dsts_ref[...]
    cond = (s == ss) & (s != d) & (ss != ds)
    rows, cols = cond.shape
    idx = (lax.broadcasted_iota(jnp.int32, (rows, cols), 0) * cols
           + lax.broadcasted_iota(jnp.int32, (rows, cols), 1))
    last = jnp.max(jnp.where(cond, idx, -1))
    m = (cond & (idx != last)).astype(F32)
    mask_ref[...] = m
    col0 = m[:, 0:1]
    col0_up = jnp.concatenate([col0[1:], jnp.zeros((1, 1), F32)], axis=0)
    mn = jnp.concatenate([m[:, 1:], col0_up], axis=1)

    vx = vx_ref[...]
    vy = vy_ref[...]
    vz = vz_ref[...]
    vxs = vxs_ref[...]
    vys = vys_ref[...]
    vzs = vzs_ref[...]
    nv = jnp.sqrt(vx * vx + vy * vy + vz * vz)
    nvs = jnp.sqrt(vxs * vxs + vys * vys + vzs * vzs)
    dot = vx * vxs + vy * vys + vz * vzs
    cos = (m * mn * dot
           / (jnp.maximum(nv, 1e-12) * jnp.maximum(nvs, 1e-12)))

    nf = 16
    acc = jnp.zeros_like(cos)
    for j in range(nf):
        t = cos * aw1_ref[0, j] + ab1_ref[0, j]
        o_ref[j] = t
        acc = acc + t
    mu = acc * (1.0 / nf)
    acc = jnp.zeros_like(cos)
    for j in range(nf):
        dj = o_ref[j] - mu
        acc = acc + dj * dj
    inv = lax.rsqrt(acc * (1.0 / nf) + 1e-6)
    for j in range(nf):
        u = (o_ref[j] - mu) * inv * ag_ref[0, j] + abt_ref[0, j]
        o_ref[j] = u * jax.nn.sigmoid(u)


def _make_cond(src2d, srcs2d, dst2d, dsts2d, v2d, vs2d, aw1, ab1, ag, abt):
    nrow, ncol = src2d.shape
    return pl.pallas_call(
        _cond_body,
        grid=(1,),
        in_specs=([pl.BlockSpec((nrow, ncol), lambda i: (0, 0))] * 10
                  + [pl.BlockSpec(memory_space=pltpu.SMEM)] * 4),
        out_specs=[
            pl.BlockSpec((nrow, ncol), lambda i: (0, 0)),
            pl.BlockSpec((16, nrow, ncol), lambda i: (0, 0, 0)),
        ],
        out_shape=[
            jax.ShapeDtypeStruct((nrow, ncol), F32),
            jax.ShapeDtypeStruct((16, nrow, ncol), F32),
        ],
    )(src2d, srcs2d, dst2d, dsts2d, *v2d, *vs2d, aw1, ab1, ag, abt)


def _tables_body(atom_ref, wp_ref, wq_ref, wr_ref, p_ref, q_ref, r_ref):
    at = atom_ref[...]
    p_ref[...] = jnp.dot(at, wp_ref[...], preferred_element_type=F32)
    q_ref[...] = jnp.dot(at, wq_ref[...], preferred_element_type=F32)
    r_ref[...] = jnp.dot(at, wr_ref[...], preferred_element_type=F32)


def _make_tables(atom, wp, wq, wr, bn):
    n, a = atom.shape
    w = wp.shape[1]
    return pl.pallas_call(
        _tables_body,
        grid=(n // bn,),
        in_specs=[
            pl.BlockSpec((bn, a), lambda i: (i, 0)),
            pl.BlockSpec((a, w), lambda i: (0, 0)),
            pl.BlockSpec((a, w), lambda i: (0, 0)),
            pl.BlockSpec((a, w), lambda i: (0, 0)),
        ],
        out_specs=[pl.BlockSpec((bn, w), lambda i: (i, 0))] * 3,
        out_shape=[jax.ShapeDtypeStruct((n, w), F32)] * 3,
    )(atom, wp, wq, wr)


def _make_gather_sum(width, e, chunk, nbuf=2):
    per_tile = e // NW
    iters = per_tile // chunk
    mesh = plsc.VectorSubcoreMesh(core_axis_name="c", subcore_axis_name="s",
                                  num_cores=NC, num_subcores=NS)

    @functools.partial(
        pl.kernel,
        out_type=jax.ShapeDtypeStruct((e, width), F32),
        mesh=mesh,
        scratch_types=(
            [pltpu.VMEM((per_tile,), jnp.int32)] * 3
            + [pltpu.VMEM((chunk, width), F32)] * (3 * nbuf)
            + [pltpu.SemaphoreType.DMA] * (2 * nbuf)
        ),
    )
    def gsum_kernel(*refs):
        tables = refs[:3]
        idxs = refs[3:6]
        out = refs[6]
        scr = refs[7:]
        idx_all = scr[:3]
        bufs = [scr[3 + 3 * j:3 + 3 * j + 3] for j in range(nbuf)]
        gsem = scr[3 + 3 * nbuf:3 + 4 * nbuf]
        wsem = scr[3 + 4 * nbuf:3 + 5 * nbuf]
        wid = lax.axis_index("s") * NC + lax.axis_index("c")
        base0 = pl.multiple_of(wid * per_tile, 8)

        for t in range(3):
            pltpu.sync_copy(idxs[t].at[pl.ds(base0, per_tile)], idx_all[t])

        def isl(t, k):
            return idx_all[t].at[pl.ds(pl.multiple_of(k * chunk, 8), chunk)]

        def start(k, j):
            for t in range(3):
                pltpu.async_copy(tables[t].at[isl(t, k)], bufs[j][t],
                                 gsem[j])

        def wait_gathers(k, j):
            for t in range(3):
                pltpu.make_async_copy(tables[t].at[isl(t, k)], bufs[j][t],
                                      gsem[j]).wait()

        def tec_sum(j):
            b0, b1, b2 = bufs[j]

            def srow(r, _):
                for c in range(width // 16):
                    cs = pl.ds(c * 16, 16)
                    b0[r, cs] = b0[r, cs] + b1[r, cs] + b2[r, cs]
                return 0

            lax.fori_loop(0, chunk, srow, 0, unroll=False)

        def start_wb(k, j):
            b = pl.multiple_of(base0 + k * chunk, 8)
            pltpu.async_copy(bufs[j][0], out.at[pl.ds(b, chunk)], wsem[j])

        def wait_wb(k, j):
            b = pl.multiple_of(base0 + k * chunk, 8)
            pltpu.make_async_copy(bufs[j][0], out.at[pl.ds(b, chunk)],
                                  wsem[j]).wait()

        for j in range(min(nbuf, iters)):
            start(j, j)

        def body(it, _):
            for j in range(nbuf):
                k = it * nbuf + j

                @pl.when(k < iters)
                def _():
                    wait_gathers(k, j)
                    tec_sum(j)
                    start_wb(k, j)

                    @pl.when(k + nbuf < iters)
                    def _():
                        wait_wb(k, j)
                        start(k + nbuf, j)
            return 0

        lax.fori_loop(0, (iters + nbuf - 1) // nbuf, body, 0, unroll=False)
        for j in range(min(nbuf, iters)):
            klast = iters - 1 - ((iters - 1 - j) % nbuf)
            wait_wb(klast, j)

    return gsum_kernel


def _make_gather_pair(hm, e, chunk, nbuf=2):
    width = 128
    per_tile = e // NW
    iters = per_tile // chunk
    mesh = plsc.VectorSubcoreMesh(core_axis_name="c", subcore_axis_name="s",
                                  num_cores=NC, num_subcores=NS)

    @functools.partial(
        pl.kernel,
        out_type=jax.ShapeDtypeStruct((e, width), F32),
        mesh=mesh,
        scratch_types=(
            [pltpu.VMEM((per_tile,), jnp.int32)] * 2
            + [pltpu.VMEM((chunk, width), F32)] * (2 * nbuf)
            + [pltpu.SemaphoreType.DMA] * (2 * nbuf)
        ),
    )
    def gpair_kernel(*refs):
        tables = refs[:2]
        idxs = refs[2:4]
        out = refs[4]
        scr = refs[5:]
        idx_all = scr[:2]
        bufs = [scr[2 + 2 * j:2 + 2 * j + 2] for j in range(nbuf)]
        gsem = scr[2 + 2 * nbuf:2 + 3 * nbuf]
        wsem = scr[2 + 3 * nbuf:2 + 4 * nbuf]
        wid = lax.axis_index("s") * NC + lax.axis_index("c")
        base0 = pl.multiple_of(wid * per_tile, 8)

        for t in range(2):
            pltpu.sync_copy(idxs[t].at[pl.ds(base0, per_tile)], idx_all[t])

        def isl(t, k):
            return idx_all[t].at[pl.ds(pl.multiple_of(k * chunk, 8), chunk)]

        def start(k, j):
            for t in range(2):
                pltpu.async_copy(tables[t].at[isl(t, k)], bufs[j][t],
                                 gsem[j])

        def wait_gathers(k, j):
            for t in range(2):
                pltpu.make_async_copy(tables[t].at[isl(t, k)], bufs[j][t],
                                      gsem[j]).wait()

        def tec_pair(j):
            b0, b1 = bufs[j]

            def srow(r, _):
                for c in range(hm // 16):
                    cs = pl.ds(hm + c * 16, 16)
                    b0[r, cs] = b1[r, cs]
                return 0

            lax.fori_loop(0, chunk, srow, 0, unroll=False)

        def start_wb(k, j):
            b = pl.multiple_of(base0 + k * chunk, 8)
            pltpu.async_copy(bufs[j][0], out.at[pl.ds(b, chunk)], wsem[j])

        def wait_wb(k, j):
            b = pl.multiple_of(base0 + k * chunk, 8)
            pltpu.make_async_copy(bufs[j][0], out.at[pl.ds(b, chunk)],
                                  wsem[j]).wait()

        for j in range(min(nbuf, iters)):
            start(j, j)

        def body(it, _):
            for j in range(nbuf):
                k = it * nbuf + j

                @pl.when(k < iters)
                def _():
                    wait_gathers(k, j)
                    tec_pair(j)
                    start_wb(k, j)

                    @pl.when(k + nbuf < iters)
                    def _():
                        wait_wb(k, j)
                        start(k + nbuf, j)
            return 0

        lax.fori_loop(0, (iters + nbuf - 1) // nbuf, body, 0, unroll=False)
        for j in range(min(nbuf, iters)):
            klast = iters - 1 - ((iters - 1 - j) % nbuf)
            wait_wb(klast, j)

    return gpair_kernel


def _make_gather(n_tables, width, e, chunk, nbuf=4, dtype=F32):
    per_tile = e // NW
    iters = per_tile // chunk
    mesh = plsc.VectorSubcoreMesh(core_axis_name="c", subcore_axis_name="s",
                                  num_cores=NC, num_subcores=NS)

    @functools.partial(
        pl.kernel,
        out_type=[jax.ShapeDtypeStruct((e, width), dtype)] * n_tables,
        mesh=mesh,
        scratch_types=(
            [pltpu.VMEM((per_tile,), jnp.int32)] * n_tables
            + [pltpu.VMEM((chunk, width), dtype)] * nbuf
            + [pltpu.SemaphoreType.DMA] * (2 * nbuf)
        ),
    )
    def gather_kernel(*refs):
        tables = refs[:n_tables]
        idxs = refs[n_tables:2 * n_tables]
        outs = refs[2 * n_tables:3 * n_tables]
        scr = refs[3 * n_tables:]
        idx_all = scr[:n_tables]
        rows_v = scr[n_tables:n_tables + nbuf]
        gsem = scr[n_tables + nbuf:n_tables + 2 * nbuf]
        wsem = scr[n_tables + 2 * nbuf:n_tables + 3 * nbuf]
        wid = lax.axis_index("s") * NC + lax.axis_index("c")
        base0 = pl.multiple_of(wid * per_tile, 8)

        for t in range(n_tables):
            pltpu.sync_copy(idxs[t].at[pl.ds(base0, per_tile)], idx_all[t])

        for t, (tbl, out) in enumerate(zip(tables, outs)):
            iall = idx_all[t]

            def islice(k):
                return iall.at[pl.ds(pl.multiple_of(k * chunk, 8), chunk)]

            def start(k, j):
                pltpu.async_copy(tbl.at[islice(k)], rows_v[j], gsem[j])

            def wait_gather(k, j):
                pltpu.make_async_copy(tbl.at[islice(k)], rows_v[j],
                                      gsem[j]).wait()

            def start_wb(k, j):
                b = pl.multiple_of(base0 + k * chunk, 8)
                pltpu.async_copy(rows_v[j], out.at[pl.ds(b, chunk)], wsem[j])

            def wait_wb(k, j):
                b = pl.multiple_of(base0 + k * chunk, 8)
                pltpu.make_async_copy(rows_v[j], out.at[pl.ds(b, chunk)],
                                      wsem[j]).wait()

            for j in range(min(nbuf, iters)):
                start(j, j)

            def body(it, _):
                for j in range(nbuf):
                    k = it * nbuf + j

                    @pl.when(k < iters)
                    def _():
                        wait_gather(k, j)
                        start_wb(k, j)

                        @pl.when(k + nbuf < iters)
                        def _():
                            wait_wb(k, j)
                            start(k + nbuf, j)
                return 0

            lax.fori_loop(0, (iters + nbuf - 1) // nbuf, body, 0,
                          unroll=False)
            for j in range(min(nbuf, iters)):
                klast = iters - 1 - ((iters - 1 - j) % nbuf)
                wait_wb(klast, j)

    return gather_kernel


def _make_scatter_add(n, width, e, chunk):
    per_core = e // NC
    per_tile = per_core // NS
    iters = per_tile // chunk
    nrows_tile = n // NS
    mesh = plsc.VectorSubcoreMesh(core_axis_name="c", subcore_axis_name="s",
                                  num_cores=NC, num_subcores=NS)

    nbuf = 2

    @functools.partial(
        pl.kernel,
        out_type=jax.ShapeDtypeStruct((NC, n, width), F32),
        mesh=mesh,
        scratch_types=(
            [pltpu.VMEM((per_tile,), jnp.int32)]
            + [pltpu.VMEM((chunk, width), F32)] * nbuf
            + [pltpu.VMEM_SHARED((n, width), F32)]
            + [pltpu.SemaphoreType.DMA] * (2 * nbuf)
        ),
    )
    def scatter_kernel(x_hbm, dst_hbm, zeros_hbm, out_hbm,
                       idx_all, rb0, rb1, acc_sh, rs0, rs1, ss0, ss1):
        rbuf = (rb0, rb1)
        rsem = (rs0, rs1)
        ssem = (ss0, ss1)
        cid = lax.axis_index("c")
        sid = lax.axis_index("s")
        r0 = pl.multiple_of(sid * nrows_tile, 8)
        pltpu.sync_copy(zeros_hbm.at[pl.ds(r0, nrows_tile)],
                        acc_sh.at[pl.ds(r0, nrows_tile)])
        base0 = pl.multiple_of(cid * per_core + sid * per_tile, 8)
        pltpu.sync_copy(dst_hbm.at[pl.ds(base0, per_tile)], idx_all)
        plsc.subcore_barrier()

        def islice(k):
            return idx_all.at[pl.ds(pl.multiple_of(k * chunk, 8), chunk)]

        if True:
            def start_read(k, j):
                b = pl.multiple_of(base0 + k * chunk, 8)
                pltpu.async_copy(x_hbm.at[pl.ds(b, chunk)], rbuf[j], rsem[j])

            def wait_read(k, j):
                b = pl.multiple_of(base0 + k * chunk, 8)
                pltpu.make_async_copy(x_hbm.at[pl.ds(b, chunk)], rbuf[j],
                                      rsem[j]).wait()

            def start_scat(k, j):
                pltpu.async_copy(rbuf[j], acc_sh.at[islice(k)], ssem[j],
                                 add=True)

            def wait_scat(k, j):
                pltpu.make_async_copy(rbuf[j], acc_sh.at[islice(k)],
                                      ssem[j]).wait()

            for j in range(min(nbuf, iters)):
                start_read(j, j)

            def body(it, _):
                for j in range(nbuf):
                    k = it * nbuf + j

                    @pl.when(k < iters)
                    def _():
                        wait_read(k, j)
                        start_scat(k, j)

                        @pl.when(k + nbuf < iters)
                        def _():
                            wait_scat(k, j)
                            start_read(k + nbuf, j)
                return 0

            lax.fori_loop(0, (iters + nbuf - 1) // nbuf, body, 0,
                          unroll=False)
            for j in range(min(nbuf, iters)):
                klast = iters - 1 - ((iters - 1 - j) % nbuf)
                wait_scat(klast, j)

        plsc.subcore_barrier()
        pltpu.sync_copy(acc_sh.at[pl.ds(r0, nrows_tile)],
                        out_hbm.at[cid].at[pl.ds(r0, nrows_tile)])

    return scatter_kernel


def _edge_body(z_ref, ele_ref, eles_ref,
               m_ref, ang_ref,
               we_ref, wes_ref, wan_ref,
               ib1_ref, ig_ref, ibt_ref,
               out_ref):
    b = m_ref.shape[0]
    hw = we_ref.shape[1]
    m = m_ref[...]

    lin = (z_ref[...][:, :hw]
           + jnp.dot(ele_ref[...], we_ref[...], preferred_element_type=F32)
           + jnp.dot(eles_ref[...], wes_ref[...], preferred_element_type=F32)
           + jnp.dot(ang_ref[...], wan_ref[...], preferred_element_type=F32))
    h = m * lin + ib1_ref[...]
    jmat = jnp.full((hw, hw), 1.0 / hw, F32)
    mu = jnp.dot(h, jmat, preferred_element_type=F32)
    s2 = jnp.dot(h * h, jmat, preferred_element_type=F32)
    var = jnp.maximum(s2 - mu * mu, 0.0)
    h = (h - mu) * lax.rsqrt(var + 1e-6) * ig_ref[...] + ibt_ref[...]
    h = h * jax.nn.sigmoid(h)
    out_ref[...] = jnp.concatenate(
        [h, jnp.ones((b, 1), F32), jnp.zeros((b, 127 - hw), F32)], axis=1)


def _node_body(pa_ref, pb_ref, iw2_ref, ib2_ref, ms_ref, md_ref,
               inv_ref, st_ref):
    hw = iw2_ref.shape[0]
    hm = ms_ref.shape[1]
    b = pa_ref.shape[0]
    acc = pa_ref[...] + pb_ref[...]
    h2s = acc[:, :hw]
    cnt = acc[:, hw:hw + 1]
    inv = (jnp.dot(h2s, iw2_ref[...], preferred_element_type=F32)
           + cnt * ib2_ref[...])
    inv_ref[...] = inv
    st_ref[...] = jnp.concatenate(
        [jnp.dot(inv, ms_ref[...], preferred_element_type=F32),
         jnp.dot(inv, md_ref[...], preferred_element_type=F32),
         jnp.zeros((b, 128 - 2 * hm), F32)], axis=1)


def _merge_body(z5_ref, ele_ref, me_ref,
                mb1_ref, mg_ref, mbt_ref, mw2_ref, mb2_ref, out_ref):
    hm = me_ref.shape[1]
    z5 = z5_ref[...]
    h = (z5[:, :hm] + z5[:, hm:2 * hm]
         + jnp.dot(ele_ref[...], me_ref[...], preferred_element_type=F32)
         + mb1_ref[...])
    jmat = jnp.full((hm, hm), 1.0 / hm, F32)
    mu = jnp.dot(h, jmat, preferred_element_type=F32)
    s2 = jnp.dot(h * h, jmat, preferred_element_type=F32)
    var = jnp.maximum(s2 - mu * mu, 0.0)
    h = (h - mu) * lax.rsqrt(var + 1e-6) * mg_ref[...] + mbt_ref[...]
    h = h * jax.nn.sigmoid(h)
    out_ref[...] = (jnp.dot(h, mw2_ref[...], preferred_element_type=F32)
                    + mb2_ref[...])


def _row(x):
    return x.reshape(1, -1)


@jax.jit
def kernel(atom_embedded, edge_length_embedded, edge_vec, edge_index,
           aw1, ab1, ag, abt, aw2, ab2,
           iw1, ib1, ig, ibt, iw2, ib2,
           mw1, mb1, mg, mbt, mw2, mb2):
    n, a = atom_embedded.shape
    e, f = edge_length_embedded.shape
    dn = iw2.shape[1]
    de = mw2.shape[1]
    h_inv = iw1.shape[1]
    waug = 128

    src = edge_index[0].astype(jnp.int32)
    dst = edge_index[1].astype(jnp.int32)
    zi = jnp.zeros((1,), jnp.int32)
    srcs = jnp.concatenate([src[1:], zi])
    dsts = jnp.concatenate([dst[1:], zi])

    w_src = iw1[0:a]
    w_dst = iw1[a:2 * a]
    w_e = iw1[2 * a:2 * a + f]
    w_dss = iw1[2 * a + f:3 * a + f]
    w_es = iw1[3 * a + f:3 * a + 2 * f]
    w_n = iw1[3 * a + 2 * f:]

    ncol = 128
    evs_full = jnp.concatenate([edge_vec[1:], jnp.zeros((1, 3), F32)], axis=0)
    v2d = [edge_vec[:, j].reshape(-1, ncol) for j in range(3)]
    vs2d = [evs_full[:, j].reshape(-1, ncol) for j in range(3)]
    mask2d, o3d = _make_cond(
        src.reshape(-1, ncol), srcs.reshape(-1, ncol),
        dst.reshape(-1, ncol), dsts.reshape(-1, ncol), v2d, vs2d,
        _row(aw1), _row(ab1), _row(ag), _row(abt))
    mask_e = mask2d.reshape(e, 1)
    angle16 = o3d.reshape(16, e).T

    zcol = jnp.zeros((a, 128 - h_inv), F32)
    p_tbl, q_tbl, r_tbl = _make_tables(
        atom_embedded,
        jnp.concatenate([w_src, zcol], axis=1),
        jnp.concatenate([w_dst, zcol], axis=1),
        jnp.concatenate([w_dss, zcol], axis=1), 2000)
    z_edge = _make_gather_sum(128, e, 40, nbuf=4)(p_tbl, q_tbl, r_tbl,
                                                  src, dst, dsts)

    ele = edge_length_embedded
    eles = jnp.concatenate([ele[1:], jnp.zeros((1, f), F32)], axis=0)

    w_an = aw2 @ w_n
    ib1_eff = _row(ib1) + _row(ab2) @ w_n

    be = 3200
    h2aug = pl.pallas_call(
        _edge_body,
        grid=(e // be,),
        in_specs=[
            pl.BlockSpec((be, 128), lambda i: (i, 0)),
            pl.BlockSpec((be, f), lambda i: (i, 0)),
            pl.BlockSpec((be, f), lambda i: (i, 0)),
            pl.BlockSpec((be, 1), lambda i: (i, 0)),
            pl.BlockSpec((be, f), lambda i: (i, 0)),
            pl.BlockSpec((f, h_inv), lambda i: (0, 0)),
            pl.BlockSpec((f, h_inv), lambda i: (0, 0)),
            pl.BlockSpec((f, h_inv), lambda i: (0, 0)),
            pl.BlockSpec((1, h_inv), lambda i: (0, 0)),
            pl.BlockSpec((1, h_inv), lambda i: (0, 0)),
            pl.BlockSpec((1, h_inv), lambda i: (0, 0)),
        ],
        out_specs=pl.BlockSpec((be, waug), lambda i: (i, 0)),
        out_shape=jax.ShapeDtypeStruct((e, waug), F32),
    )(z_edge, ele, eles, mask_e, angle16,
      w_e, w_es, w_an,
      ib1_eff, _row(ig), _row(ibt))

    npad = 10240
    zeros_init = jnp.zeros((npad, waug), F32)
    partials = _make_scatter_add(npad, waug, e, 40)(h2aug, dst, zeros_init)
    pa, pb = partials[0], partials[1]

    h_mrg = mw1.shape[1]
    m_s = mw1[0:dn]
    m_d = mw1[dn:2 * dn]
    m_e = mw1[2 * dn:]

    bn2 = 2048
    inv_pad, st_tbl = pl.pallas_call(
        _node_body,
        grid=(npad // bn2,),
        in_specs=[
            pl.BlockSpec((bn2, waug), lambda i: (i, 0)),
            pl.BlockSpec((bn2, waug), lambda i: (i, 0)),
            pl.BlockSpec((h_inv, dn), lambda i: (0, 0)),
            pl.BlockSpec((1, dn), lambda i: (0, 0)),
            pl.BlockSpec((dn, h_mrg), lambda i: (0, 0)),
            pl.BlockSpec((dn, h_mrg), lambda i: (0, 0)),
        ],
        out_specs=[
            pl.BlockSpec((bn2, dn), lambda i: (i, 0)),
            pl.BlockSpec((bn2, 128), lambda i: (i, 0)),
        ],
        out_shape=[
            jax.ShapeDtypeStruct((npad, dn), F32),
            jax.ShapeDtypeStruct((npad, 128), F32),
        ],
    )(pa, pb, iw2, _row(ib2), m_s, m_d)
    inv_node = inv_pad[:n]

    z5 = _make_gather_pair(h_mrg, e, 200)(st_tbl, st_tbl, src, dst)

    inv_fea_edge = pl.pallas_call(
        _merge_body,
        grid=(e // be,),
        in_specs=[
            pl.BlockSpec((be, 128), lambda i: (i, 0)),
            pl.BlockSpec((be, f), lambda i: (i, 0)),
            pl.BlockSpec((f, h_mrg), lambda i: (0, 0)),
            pl.BlockSpec((1, h_mrg), lambda i: (0, 0)),
            pl.BlockSpec((1, h_mrg), lambda i: (0, 0)),
            pl.BlockSpec((1, h_mrg), lambda i: (0, 0)),
            pl.BlockSpec((h_mrg, de), lambda i: (0, 0)),
            pl.BlockSpec((1, de), lambda i: (0, 0)),
        ],
        out_specs=pl.BlockSpec((be, de), lambda i: (i, 0)),
        out_shape=jax.ShapeDtypeStruct((e, de), F32),
    )(z5, ele, m_e,
      _row(mb1), _row(mg), _row(mbt), mw2, _row(mb2))

    return (inv_node, inv_fea_edge)
